# Initial kernel scaffold; baseline (speedup 1.0000x reference)
import jax
import jax.numpy as jnp
from jax.experimental import pallas as pl


def kernel(x, edge_index, W0, Wk0, Wq0, SW0, Sb0, g0, b0, W1, Wk1, Wq1, SW1, Sb1, g1, b1):
    raise NotImplementedError("write your pallas kernel here")


# speedup vs baseline: 1.3292x; 1.3292x over previous
import functools
import jax
import jax.numpy as jnp
from jax import lax
from jax.experimental import pallas as pl
from jax.experimental.pallas import tpu as pltpu
from jax.experimental.pallas import tpu_sc as plsc

N = 10000
E = 160000
D = 128
HID = 256
HEADS = 4
KQ = 256
HIN = HEADS * HID
F32 = jnp.float32

NC = 2
NS = 16
NW = NC * NS

RB = 1000
EPW = E // NW
BB = 40
EPC = E // NS
BC = 16
NGRP = 8
GPC = NGRP // NC

_mesh = plsc.VectorSubcoreMesh(core_axis_name="c", subcore_axis_name="s")
_SC_PARAMS = pltpu.CompilerParams(use_tc_tiling_on_sc=False,
                                  needs_layout_passes=False)


def _a1_body(x_ref, w_ref, g_ref, b_ref, sb_ref, wp_ref, bp_ref):
    x = x_ref[...]
    mu = jnp.mean(x, axis=0, keepdims=True)
    var = jnp.mean(x * x, axis=0, keepdims=True) - mu * mu
    a = g_ref[...] * lax.rsqrt(var + 1e-5)
    c = b_ref[...] - mu * a
    w = w_ref[...]
    wp_ref[...] = w * jnp.transpose(a)
    bp_ref[...] = jnp.dot(c, w, preferred_element_type=F32, precision=lax.Precision.HIGHEST) + sb_ref[...]


def _a1_call(x, wcat, g, b, sbpad):
    return pl.pallas_call(
        _a1_body,
        out_shape=[
            jax.ShapeDtypeStruct((D, 4 * HIN), F32),
            jax.ShapeDtypeStruct((1, 4 * HIN), F32),
        ],
    )(x, wcat, g, b, sbpad)


def _a2_body(x_ref, wp_ref, bp_ref, h_ref, kn_ref, qn_ref, s_ref):
    y = jnp.dot(x_ref[...], wp_ref[...], preferred_element_type=F32, precision=lax.Precision.HIGHEST) + bp_ref[...]
    h_ref[...] = y[:, 0:HIN]
    s_ref[...] = y[:, 3 * HIN:4 * HIN]
    for lo, ref in ((HIN, kn_ref), (2 * HIN, qn_ref)):
        for hh in range(HEADS):
            ch = y[:, lo + hh * KQ: lo + (hh + 1) * KQ]
            nrm = jnp.sqrt(jnp.sum(ch * ch, axis=1, keepdims=True)) + 1e-8
            ref[:, hh * KQ:(hh + 1) * KQ] = ch / nrm


def _a2_call(x, wp, bp):
    nsteps = N // RB
    return pl.pallas_call(
        _a2_body,
        grid=(nsteps,),
        in_specs=[
            pl.BlockSpec((RB, D), lambda i: (i, 0)),
            pl.BlockSpec((D, 4 * HIN), lambda i: (0, 0)),
            pl.BlockSpec((1, 4 * HIN), lambda i: (0, 0)),
        ],
        out_specs=[
            pl.BlockSpec((RB, HIN), lambda i: (i, 0)),
            pl.BlockSpec((RB, HIN), lambda i: (i, 0)),
            pl.BlockSpec((RB, HIN), lambda i: (i, 0)),
            pl.BlockSpec((RB, HIN), lambda i: (i, 0)),
        ],
        out_shape=[jax.ShapeDtypeStruct((N, HIN), F32)] * 4,
    )(x, wp, bp)


EB_LO = 4992
EB_HI = 5008


def _worker_span(wid):
    extra = jnp.minimum(wid, 16) * 16
    base = wid * EB_LO + extra
    nb = jnp.where(wid < 16, EB_HI // 16, EB_LO // 16)
    return base, nb


def _b_body(src_ref, dst_ref, kn_ref, qn_ref, z_ref, p_ref, den_ref,
            ebs, ebd, kbuf, qbuf, pbuf, pall, dacc, sem, sem2):
    ci = lax.axis_index("c")
    si = lax.axis_index("s")
    wid = si * NC + ci

    @pl.when(si == 0)
    def _():
        pltpu.sync_copy(z_ref, dacc)

    zero16 = jnp.zeros((16,), F32)
    for e in range(16):
        pbuf[e, :] = zero16
    plsc.subcore_barrier()

    base_w, nb = _worker_span(wid)
    rows = lax.iota(jnp.int32, 16)

    def batch_body(bi, carry):
        base = base_w + bi * 16
        pltpu.sync_copy(src_ref.at[pl.ds(base, 16)], ebs)
        pltpu.sync_copy(dst_ref.at[pl.ds(base, 16)], ebd)
        cp1 = pltpu.async_copy(kn_ref.at[ebs], kbuf, sem)
        cp2 = pltpu.async_copy(qn_ref.at[ebd], qbuf, sem2)
        cp1.wait()
        cp2.wait()

        def col_body(j, accs):
            colj = jnp.full((16,), 0, jnp.int32) + j
            new = []
            for hh in range(HEADS):
                cols = colj + (hh * KQ)
                ck = plsc.load_gather(kbuf, [rows, cols])
                cq = plsc.load_gather(qbuf, [rows, cols])
                new.append(accs[hh] + ck * cq)
            return tuple(new)

        accs = lax.fori_loop(0, KQ, col_body,
                             tuple(jnp.zeros((16,), F32) for _ in range(HEADS)),
                             unroll=4)
        for hh in range(HEADS):
            pv = jnp.exp(accs[hh] * (1.0 / 16.0))
            pall[hh, pl.ds(bi * 16, 16)] = pv
            plsc.store_scatter(pbuf, [rows, jnp.full((16,), hh, jnp.int32)], pv)
        pltpu.sync_copy(pbuf, dacc.at[ebd], add=True)
        return carry

    lax.fori_loop(0, nb, batch_body, 0)

    @pl.when(wid < 16)
    def _():
        for hh in range(HEADS):
            pltpu.sync_copy(pall.at[hh, pl.ds(0, EB_HI)],
                            p_ref.at[pl.ds(hh * E + base_w, EB_HI)])

    @pl.when(wid >= 16)
    def _():
        for hh in range(HEADS):
            pltpu.sync_copy(pall.at[hh, pl.ds(0, EB_LO)],
                            p_ref.at[pl.ds(hh * E + base_w, EB_LO)])

    plsc.subcore_barrier()

    @pl.when(si == 0)
    def _():
        pltpu.sync_copy(dacc, den_ref.at[ci])


def _b_call(src, dst, kn, qn, zn16):
    f = functools.partial(
        pl.kernel,
        out_type=[
            jax.ShapeDtypeStruct((HEADS * E,), F32),
            jax.ShapeDtypeStruct((NC, N, 16), F32),
        ],
        mesh=_mesh,
        compiler_params=_SC_PARAMS,
        scratch_types=[
            pltpu.VMEM((16,), jnp.int32),
            pltpu.VMEM((16,), jnp.int32),
            pltpu.VMEM((16, HEADS * KQ), F32),
            pltpu.VMEM((16, HEADS * KQ), F32),
            pltpu.VMEM((16, 16), F32),
            pltpu.VMEM((HEADS, EB_HI), F32),
            pltpu.VMEM_SHARED((N, 16), F32),
            pltpu.SemaphoreType.DMA,
            pltpu.SemaphoreType.DMA,
        ],
    )(_b_body)
    return f(src, dst, kn, qn, zn16)


def _c_body(src_ref, dst_ref, p_ref, h8_ref, z_ref, g_out,
            ebs, ebd, pb, idx8, hbuf, gacc, sem):
    ci = lax.axis_index("c")
    si = lax.axis_index("s")
    for gi in range(GPC):
        g = ci * GPC + gi
        head = g // 2

        @pl.when(si == 0)
        def _():
            pltpu.sync_copy(z_ref, gacc)

        plsc.subcore_barrier()

        rows = lax.iota(jnp.int32, 16)

        def batch_body(bi, carry):
            base = si * EPC + bi * BC
            pltpu.sync_copy(src_ref.at[pl.ds(base, BC)], ebs)
            pltpu.sync_copy(dst_ref.at[pl.ds(base, BC)], ebd)
            pltpu.sync_copy(p_ref.at[pl.ds(head * E + base, BC)], pb)
            idx8[:] = ebs[:] * 8 + g
            pltpu.async_copy(h8_ref.at[idx8], hbuf, sem).wait()
            pvec = pb[:]

            def col_body(c, c2):
                cc = jnp.full((16,), 0, jnp.int32) + c
                v = plsc.load_gather(hbuf, [rows, cc])
                plsc.store_scatter(hbuf, [rows, cc], v * pvec)
                return c2

            lax.fori_loop(0, D, col_body, 0, unroll=8)
            pltpu.sync_copy(hbuf, gacc.at[ebd], add=True)
            return carry

        lax.fori_loop(0, EPC // BC, batch_body, 0)
        plsc.subcore_barrier()

        @pl.when(si == 0)
        def _():
            pltpu.sync_copy(gacc, g_out.at[g])

        plsc.subcore_barrier()


def _c_call(src, dst, p, h8, zn128):
    f = functools.partial(
        pl.kernel,
        out_type=jax.ShapeDtypeStruct((NGRP, N, D), F32),
        mesh=_mesh,
        compiler_params=_SC_PARAMS,
        scratch_types=[
            pltpu.VMEM((BC,), jnp.int32),
            pltpu.VMEM((BC,), jnp.int32),
            pltpu.VMEM((BC,), F32),
            pltpu.VMEM((BC,), jnp.int32),
            pltpu.VMEM((BC, D), F32),
            pltpu.VMEM_SHARED((N, D), F32),
            pltpu.SemaphoreType.DMA,
        ],
    )(_c_body)
    return f(src, dst, p, h8, zn128)


def _d_body(s_ref, g_ref, den_ref, y_ref, cs_ref, cq_ref):
    g = pl.program_id(0)
    i = pl.program_id(1)
    head = g // 2
    den = den_ref[0] + den_ref[1]
    onehot = (lax.broadcasted_iota(jnp.int32, (1, 16), 1) == head).astype(F32)
    dh = jnp.sum(den * onehot, axis=1, keepdims=True)
    rec = 1.0 / (dh + 1e-16)
    y = jnp.maximum(s_ref[...] + g_ref[0] * rec, 0.0)
    y_ref[...] = y

    @pl.when(i == 0)
    def _():
        cs_ref[...] = jnp.zeros_like(cs_ref)
        cq_ref[...] = jnp.zeros_like(cq_ref)

    cs_ref[...] += jnp.sum(y, axis=0, keepdims=True)
    cq_ref[...] += jnp.sum(y * y, axis=0, keepdims=True)


def _d_call(s, gacc, den):
    nsteps = N // RB
    return pl.pallas_call(
        _d_body,
        grid=(NGRP, nsteps),
        in_specs=[
            pl.BlockSpec((RB, D), lambda g, i: (i, g)),
            pl.BlockSpec((1, RB, D), lambda g, i: (g, i, 0)),
            pl.BlockSpec((NC, RB, 16), lambda g, i: (0, i, 0)),
        ],
        out_specs=[
            pl.BlockSpec((RB, D), lambda g, i: (i, g)),
            pl.BlockSpec((1, D), lambda g, i: (0, g)),
            pl.BlockSpec((1, D), lambda g, i: (0, g)),
        ],
        out_shape=[
            jax.ShapeDtypeStruct((N, HIN), F32),
            jax.ShapeDtypeStruct((1, HIN), F32),
            jax.ShapeDtypeStruct((1, HIN), F32),
        ],
    )(s, gacc, den)


def _d2_body(cs_ref, cq_ref, g_ref, b_ref, w_ref, sb_ref, wp_ref, bp_ref):
    mu = cs_ref[...] * (1.0 / N)
    var = cq_ref[...] * (1.0 / N) - mu * mu
    a = g_ref[...] * lax.rsqrt(var + 1e-5)
    c = b_ref[...] - mu * a
    w = w_ref[...]
    wp_ref[...] = w * jnp.transpose(a)
    bp_ref[...] = jnp.dot(c, w, preferred_element_type=F32, precision=lax.Precision.HIGHEST) + sb_ref[...]


def _d2_call(cs, cq, g, b, wcat1, sb1pad):
    return pl.pallas_call(
        _d2_body,
        out_shape=[
            jax.ShapeDtypeStruct((HIN, 516), F32),
            jax.ShapeDtypeStruct((1, 516), F32),
        ],
    )(cs, cq, g, b, wcat1, sb1pad)


def _d3_body(y_ref, wp_ref, bp_ref, hs_ref, k_ref, q_ref):
    y = jnp.dot(y_ref[...], wp_ref[...], preferred_element_type=F32, precision=lax.Precision.HIGHEST) + bp_ref[...]
    k = y[:, 0:KQ]
    q = y[:, KQ:2 * KQ]
    hs_ref[...] = jnp.concatenate(
        [y[:, 2 * KQ:2 * KQ + 4], jnp.zeros((y.shape[0], 12), F32)], axis=1)
    k_ref[...] = k / (jnp.sqrt(jnp.sum(k * k, axis=1, keepdims=True)) + 1e-8)
    q_ref[...] = q / (jnp.sqrt(jnp.sum(q * q, axis=1, keepdims=True)) + 1e-8)


def _d3_call(y0, wp1, bp1):
    nsteps = N // RB
    return pl.pallas_call(
        _d3_body,
        grid=(nsteps,),
        in_specs=[
            pl.BlockSpec((RB, HIN), lambda i: (i, 0)),
            pl.BlockSpec((HIN, 516), lambda i: (0, 0)),
            pl.BlockSpec((1, 516), lambda i: (0, 0)),
        ],
        out_specs=[
            pl.BlockSpec((RB, 16), lambda i: (i, 0)),
            pl.BlockSpec((RB, KQ), lambda i: (i, 0)),
            pl.BlockSpec((RB, KQ), lambda i: (i, 0)),
        ],
        out_shape=[
            jax.ShapeDtypeStruct((N, 16), F32),
            jax.ShapeDtypeStruct((N, KQ), F32),
            jax.ShapeDtypeStruct((N, KQ), F32),
        ],
    )(y0, wp1, bp1)


def _e_body(src_ref, dst_ref, k_ref, q_ref, hs_ref, z_ref, ep_ref,
            ebs, ebd, kbuf, qbuf, hbuf, mbuf, macc, sem, sem2, sem3):
    ci = lax.axis_index("c")
    si = lax.axis_index("s")
    wid = si * NC + ci

    @pl.when(si == 0)
    def _():
        pltpu.sync_copy(z_ref, macc)

    zero16 = jnp.zeros((16,), F32)
    for e in range(16):
        mbuf[e, :] = zero16
    plsc.subcore_barrier()

    base_w, nb = _worker_span(wid)
    rows = lax.iota(jnp.int32, 16)

    def batch_body(bi, carry):
        base = base_w + bi * 16
        pltpu.sync_copy(src_ref.at[pl.ds(base, 16)], ebs)
        pltpu.sync_copy(dst_ref.at[pl.ds(base, 16)], ebd)
        cp1 = pltpu.async_copy(k_ref.at[ebs], kbuf, sem)
        cp2 = pltpu.async_copy(q_ref.at[ebd], qbuf, sem2)
        cp3 = pltpu.async_copy(hs_ref.at[ebs], hbuf, sem3)
        cp1.wait()
        cp2.wait()
        cp3.wait()

        def col_body(j, acc):
            colj = jnp.full((16,), 0, jnp.int32) + j
            ck = plsc.load_gather(kbuf, [rows, colj])
            cq = plsc.load_gather(qbuf, [rows, colj])
            return acc + ck * cq

        acc = lax.fori_loop(0, KQ, col_body, jnp.zeros((16,), F32), unroll=8)
        pv = jnp.exp(acc * (1.0 / 16.0))
        h0 = plsc.load_gather(hbuf, [rows, jnp.full((16,), 0, jnp.int32)])
        h1 = plsc.load_gather(hbuf, [rows, jnp.full((16,), 1, jnp.int32)])
        plsc.store_scatter(mbuf, [rows, jnp.full((16,), 0, jnp.int32)], pv * h0)
        plsc.store_scatter(mbuf, [rows, jnp.full((16,), 1, jnp.int32)], pv * h1)
        plsc.store_scatter(mbuf, [rows, jnp.full((16,), 2, jnp.int32)], pv)
        pltpu.sync_copy(mbuf, macc.at[ebd], add=True)
        return carry

    lax.fori_loop(0, nb, batch_body, 0)
    plsc.subcore_barrier()

    @pl.when(si == 0)
    def _():
        pltpu.sync_copy(macc, ep_ref.at[ci])


def _e_call(src, dst, k1, q1, hs1, zn16):
    f = functools.partial(
        pl.kernel,
        out_type=jax.ShapeDtypeStruct((NC, N, 16), F32),
        mesh=_mesh,
        compiler_params=_SC_PARAMS,
        scratch_types=[
            pltpu.VMEM((16,), jnp.int32),
            pltpu.VMEM((16,), jnp.int32),
            pltpu.VMEM((16, KQ), F32),
            pltpu.VMEM((16, KQ), F32),
            pltpu.VMEM((16, 16), F32),
            pltpu.VMEM((16, 16), F32),
            pltpu.VMEM_SHARED((N, 16), F32),
            pltpu.SemaphoreType.DMA,
            pltpu.SemaphoreType.DMA,
            pltpu.SemaphoreType.DMA,
        ],
    )(_e_body)
    return f(src, dst, k1, q1, hs1, zn16)


def _f_body(ep_ref, hs_ref, o_ref):
    m = ep_ref[0] + ep_ref[1]
    rec = 1.0 / (m[:, 2:3] + 1e-16)
    o_ref[...] = jnp.maximum(hs_ref[:, 2:4] + m[:, 0:2] * rec, 0.0)


def _f_call(ep, hs1):
    return pl.pallas_call(
        _f_body,
        out_shape=jax.ShapeDtypeStruct((N, 2), F32),
    )(ep, hs1)


def kernel(x, edge_index, W0, Wk0, Wq0, SW0, Sb0, g0, b0,
           W1, Wk1, Wq1, SW1, Sb1, g1, b1):
    src = edge_index[0]
    dst = edge_index[1]
    zn16 = jnp.zeros((N, 16), F32)
    zn128 = jnp.zeros((N, D), F32)

    wcat = jnp.concatenate([W0, Wk0, Wq0, SW0], axis=1)
    sbpad = jnp.concatenate([jnp.zeros((3 * HIN,), F32), Sb0]).reshape(1, 4 * HIN)
    wp, bp = _a1_call(x, wcat, g0.reshape(1, D), b0.reshape(1, D), sbpad)
    h, kn, qn, s = _a2_call(x, wp, bp)

    p, den = _b_call(src, dst, kn, qn, zn16)
    gacc = _c_call(src, dst, p, h.reshape(N * NGRP, D), zn128)

    y0, cs, cq = _d_call(s, gacc, den)
    wcat1 = jnp.concatenate([Wk1, Wq1, W1, SW1], axis=1)
    sb1pad = jnp.concatenate([jnp.zeros((2 * KQ + 2,), F32), Sb1]).reshape(1, 516)
    wp1, bp1 = _d2_call(cs, cq, g1.reshape(1, HIN), b1.reshape(1, HIN),
                        wcat1, sb1pad)
    hs1, k1, q1 = _d3_call(y0, wp1, bp1)

    ep = _e_call(src, dst, k1, q1, hs1, zn16)
    return _f_call(ep, hs1)


# speedup vs baseline: 1.7205x; 1.2944x over previous
import functools
import jax
import jax.numpy as jnp
from jax import lax
from jax.experimental import pallas as pl
from jax.experimental.pallas import tpu as pltpu
from jax.experimental.pallas import tpu_sc as plsc

N = 10000
E = 160000
D = 128
HID = 256
HEADS = 4
KQ = 256
HIN = HEADS * HID
F32 = jnp.float32

NC = 2
NS = 16
NW = NC * NS

RB = 1000
EPW = E // NW
BB = 40
EPC = E // NS
BC = 16
NGRP = 8
GPC = NGRP // NC

_mesh = plsc.VectorSubcoreMesh(core_axis_name="c", subcore_axis_name="s")
_SC_PARAMS = pltpu.CompilerParams(use_tc_tiling_on_sc=False,
                                  needs_layout_passes=False)


def _a1_body(x_ref, w_ref, g_ref, b_ref, sb_ref, wp_ref, bp_ref):
    x = x_ref[...]
    mu = jnp.mean(x, axis=0, keepdims=True)
    var = jnp.mean(x * x, axis=0, keepdims=True) - mu * mu
    a = g_ref[...] * lax.rsqrt(var + 1e-5)
    c = b_ref[...] - mu * a
    w = w_ref[...]
    wp_ref[...] = w * jnp.transpose(a)
    bp_ref[...] = jnp.dot(c, w, preferred_element_type=F32, precision=lax.Precision.HIGHEST) + sb_ref[...]


def _a1_call(x, wcat, g, b, sbpad):
    return pl.pallas_call(
        _a1_body,
        out_shape=[
            jax.ShapeDtypeStruct((D, 4 * HIN), F32),
            jax.ShapeDtypeStruct((1, 4 * HIN), F32),
        ],
    )(x, wcat, g, b, sbpad)


def _a2_body(x_ref, wp_ref, bp_ref, h_ref, kn_ref, qn_ref, s_ref):
    y = jnp.dot(x_ref[...], wp_ref[...], preferred_element_type=F32, precision=lax.Precision.HIGHEST) + bp_ref[...]
    h_ref[...] = y[:, 0:HIN]
    s_ref[...] = y[:, 3 * HIN:4 * HIN]
    for lo, ref in ((HIN, kn_ref), (2 * HIN, qn_ref)):
        for hh in range(HEADS):
            ch = y[:, lo + hh * KQ: lo + (hh + 1) * KQ]
            nrm = jnp.sqrt(jnp.sum(ch * ch, axis=1, keepdims=True)) + 1e-8
            ref[:, hh * KQ:(hh + 1) * KQ] = ch / nrm


def _a2_call(x, wp, bp):
    nsteps = N // RB
    return pl.pallas_call(
        _a2_body,
        grid=(nsteps,),
        in_specs=[
            pl.BlockSpec((RB, D), lambda i: (i, 0)),
            pl.BlockSpec((D, 4 * HIN), lambda i: (0, 0)),
            pl.BlockSpec((1, 4 * HIN), lambda i: (0, 0)),
        ],
        out_specs=[
            pl.BlockSpec((RB, HIN), lambda i: (i, 0)),
            pl.BlockSpec((RB, HIN), lambda i: (i, 0)),
            pl.BlockSpec((RB, HIN), lambda i: (i, 0)),
            pl.BlockSpec((RB, HIN), lambda i: (i, 0)),
        ],
        out_shape=[jax.ShapeDtypeStruct((N, HIN), F32)] * 4,
    )(x, wp, bp)


EB_LO = 4992
EB_HI = 5008


def _worker_span(wid):
    extra = jnp.minimum(wid, 16) * 16
    base = wid * EB_LO + extra
    nb = jnp.where(wid < 16, EB_HI // 16, EB_LO // 16)
    return base, nb


def _b_body(src_ref, dst_ref, kn_ref, qn_ref, z_ref, p_ref, den_ref,
            ebs, ebd, kbuf, qbuf, pbuf, pall, dacc, sem, sem2):
    ci = lax.axis_index("c")
    si = lax.axis_index("s")
    wid = si * NC + ci

    @pl.when(si == 0)
    def _():
        pltpu.sync_copy(z_ref, dacc)

    zero16 = jnp.zeros((16,), F32)
    for e in range(16):
        pbuf[e, :] = zero16
    plsc.subcore_barrier()

    base_w, nb = _worker_span(wid)
    rows = lax.iota(jnp.int32, 16)

    def batch_body(bi, carry):
        base = base_w + bi * 16
        pltpu.sync_copy(src_ref.at[pl.ds(base, 16)], ebs)
        pltpu.sync_copy(dst_ref.at[pl.ds(base, 16)], ebd)
        cp1 = pltpu.async_copy(kn_ref.at[ebs], kbuf, sem)
        cp2 = pltpu.async_copy(qn_ref.at[ebd], qbuf, sem2)
        cp1.wait()
        cp2.wait()

        def col_body(j, accs):
            colj = jnp.full((16,), 0, jnp.int32) + j
            new = []
            for hh in range(HEADS):
                cols = colj + (hh * KQ)
                ck = plsc.load_gather(kbuf, [rows, cols])
                cq = plsc.load_gather(qbuf, [rows, cols])
                new.append(accs[hh] + ck * cq)
            return tuple(new)

        accs = lax.fori_loop(0, KQ, col_body,
                             tuple(jnp.zeros((16,), F32) for _ in range(HEADS)),
                             unroll=4)
        for hh in range(HEADS):
            pv = jnp.exp(accs[hh] * (1.0 / 16.0))
            pall[hh, pl.ds(bi * 16, 16)] = pv
            plsc.store_scatter(pbuf, [rows, jnp.full((16,), hh, jnp.int32)], pv)
        pltpu.sync_copy(pbuf, dacc.at[ebd], add=True)
        return carry

    lax.fori_loop(0, nb, batch_body, 0)

    @pl.when(wid < 16)
    def _():
        for hh in range(HEADS):
            pltpu.sync_copy(pall.at[hh, pl.ds(0, EB_HI)],
                            p_ref.at[pl.ds(hh * E + base_w, EB_HI)])

    @pl.when(wid >= 16)
    def _():
        for hh in range(HEADS):
            pltpu.sync_copy(pall.at[hh, pl.ds(0, EB_LO)],
                            p_ref.at[pl.ds(hh * E + base_w, EB_LO)])

    plsc.subcore_barrier()

    @pl.when(si == 0)
    def _():
        pltpu.sync_copy(dacc, den_ref.at[ci])


def _b_call(src, dst, kn, qn, zn16):
    f = functools.partial(
        pl.kernel,
        out_type=[
            jax.ShapeDtypeStruct((HEADS * E,), F32),
            jax.ShapeDtypeStruct((NC, N, 16), F32),
        ],
        mesh=_mesh,
        compiler_params=_SC_PARAMS,
        scratch_types=[
            pltpu.VMEM((16,), jnp.int32),
            pltpu.VMEM((16,), jnp.int32),
            pltpu.VMEM((16, HEADS * KQ), F32),
            pltpu.VMEM((16, HEADS * KQ), F32),
            pltpu.VMEM((16, 16), F32),
            pltpu.VMEM((HEADS, EB_HI), F32),
            pltpu.VMEM_SHARED((N, 16), F32),
            pltpu.SemaphoreType.DMA,
            pltpu.SemaphoreType.DMA,
        ],
    )(_b_body)
    return f(src, dst, kn, qn, zn16)


CB = 80
CROWS = E // CB // NS
CCH = 25


def _c_scale(hb, pb2, bi, rows_list):
    pvs = [pb2[bi, pl.ds(r * 16, 16)] for r in range(5)]

    def col_body(c, c2):
        cc = jnp.full((16,), 0, jnp.int32) + c
        for r in range(5):
            v = plsc.load_gather(hb, [rows_list[r], cc])
            plsc.store_scatter(hb, [rows_list[r], cc], v * pvs[r])
        return c2

    lax.fori_loop(0, D, col_body, 0, unroll=4)


def _c_body(src_ref, dst_ref, p_ref, h8_ref, z_ref, g_out,
            ebd2, pb2, idx82, hb0, hb1, gacc, sem0, sem1, ssem0, ssem1):
    ci = lax.axis_index("c")
    si = lax.axis_index("s")
    rb = si * CROWS
    rows_list = [lax.iota(jnp.int32, 16) + r * 16 for r in range(5)]
    for gi in range(GPC):
        g = ci * GPC + gi
        head = g // 2

        @pl.when(si == 0)
        def _():
            pltpu.sync_copy(z_ref, gacc)

        plsc.subcore_barrier()

        def chunk_body(ch, c0):
            crb = rb + ch * CCH
            pltpu.sync_copy(src_ref.at[pl.ds(crb, CCH), :], idx82)
            pltpu.sync_copy(dst_ref.at[pl.ds(crb, CCH), :], ebd2)
            pltpu.sync_copy(p_ref.at[pl.ds(head * (NS * CROWS) + crb, CCH), :],
                            pb2)

            def idx_body(r, c2):
                for cc in range(5):
                    sl = pl.ds(cc * 16, 16)
                    idx82[r, sl] = idx82[r, sl] * 8 + g
                return c2

            lax.fori_loop(0, CCH, idx_body, 0)

            def pair_body(k, c2):
                b0 = 2 * k
                b1 = 2 * k + 1
                cpa = pltpu.async_copy(h8_ref.at[idx82.at[b0]], hb0, sem0)
                cpb = pltpu.async_copy(h8_ref.at[idx82.at[b1]], hb1, sem1)
                cpa.wait()
                _c_scale(hb0, pb2, b0, rows_list)
                sca = pltpu.async_copy(hb0, gacc.at[ebd2.at[b0]], ssem0,
                                       add=True)
                cpb.wait()
                _c_scale(hb1, pb2, b1, rows_list)
                scb = pltpu.async_copy(hb1, gacc.at[ebd2.at[b1]], ssem1,
                                       add=True)
                sca.wait()
                scb.wait()
                return c2

            lax.fori_loop(0, CCH // 2, pair_body, 0)
            bt = CCH - 1
            pltpu.async_copy(h8_ref.at[idx82.at[bt]], hb0, sem0).wait()
            _c_scale(hb0, pb2, bt, rows_list)
            pltpu.sync_copy(hb0, gacc.at[ebd2.at[bt]], add=True)
            return c0

        lax.fori_loop(0, CROWS // CCH, chunk_body, 0)
        plsc.subcore_barrier()

        @pl.when(si == 0)
        def _():
            pltpu.sync_copy(gacc, g_out.at[g])

        plsc.subcore_barrier()


def _c_call(src, dst, p, h8, zn128):
    f = functools.partial(
        pl.kernel,
        out_type=jax.ShapeDtypeStruct((NGRP, N, D), F32),
        mesh=_mesh,
        compiler_params=_SC_PARAMS,
        scratch_types=[
            pltpu.VMEM((CCH, CB), jnp.int32),
            pltpu.VMEM((CCH, CB), F32),
            pltpu.VMEM((CCH, CB), jnp.int32),
            pltpu.VMEM((CB, D), F32),
            pltpu.VMEM((CB, D), F32),
            pltpu.VMEM_SHARED((N, D), F32),
            pltpu.SemaphoreType.DMA,
            pltpu.SemaphoreType.DMA,
            pltpu.SemaphoreType.DMA,
            pltpu.SemaphoreType.DMA,
        ],
    )(_c_body)
    return f(src.reshape(E // CB, CB), dst.reshape(E // CB, CB),
             p.reshape(HEADS * (E // CB), CB), h8, zn128)


def _d_body(s_ref, g_ref, den_ref, y_ref, cs_ref, cq_ref):
    g = pl.program_id(0)
    i = pl.program_id(1)
    head = g // 2
    den = den_ref[0] + den_ref[1]
    onehot = (lax.broadcasted_iota(jnp.int32, (1, 16), 1) == head).astype(F32)
    dh = jnp.sum(den * onehot, axis=1, keepdims=True)
    rec = 1.0 / (dh + 1e-16)
    y = jnp.maximum(s_ref[...] + g_ref[0] * rec, 0.0)
    y_ref[...] = y

    @pl.when(i == 0)
    def _():
        cs_ref[...] = jnp.zeros_like(cs_ref)
        cq_ref[...] = jnp.zeros_like(cq_ref)

    cs_ref[...] += jnp.sum(y, axis=0, keepdims=True)
    cq_ref[...] += jnp.sum(y * y, axis=0, keepdims=True)


def _d_call(s, gacc, den):
    nsteps = N // RB
    return pl.pallas_call(
        _d_body,
        grid=(NGRP, nsteps),
        in_specs=[
            pl.BlockSpec((RB, D), lambda g, i: (i, g)),
            pl.BlockSpec((1, RB, D), lambda g, i: (g, i, 0)),
            pl.BlockSpec((NC, RB, 16), lambda g, i: (0, i, 0)),
        ],
        out_specs=[
            pl.BlockSpec((RB, D), lambda g, i: (i, g)),
            pl.BlockSpec((1, D), lambda g, i: (0, g)),
            pl.BlockSpec((1, D), lambda g, i: (0, g)),
        ],
        out_shape=[
            jax.ShapeDtypeStruct((N, HIN), F32),
            jax.ShapeDtypeStruct((1, HIN), F32),
            jax.ShapeDtypeStruct((1, HIN), F32),
        ],
    )(s, gacc, den)


def _d2_body(cs_ref, cq_ref, g_ref, b_ref, w_ref, sb_ref, wp_ref, bp_ref):
    mu = cs_ref[...] * (1.0 / N)
    var = cq_ref[...] * (1.0 / N) - mu * mu
    a = g_ref[...] * lax.rsqrt(var + 1e-5)
    c = b_ref[...] - mu * a
    w = w_ref[...]
    wp_ref[...] = w * jnp.transpose(a)
    bp_ref[...] = jnp.dot(c, w, preferred_element_type=F32, precision=lax.Precision.HIGHEST) + sb_ref[...]


def _d2_call(cs, cq, g, b, wcat1, sb1pad):
    return pl.pallas_call(
        _d2_body,
        out_shape=[
            jax.ShapeDtypeStruct((HIN, 516), F32),
            jax.ShapeDtypeStruct((1, 516), F32),
        ],
    )(cs, cq, g, b, wcat1, sb1pad)


def _d3_body(y_ref, wp_ref, bp_ref, hs_ref, k_ref, q_ref):
    y = jnp.dot(y_ref[...], wp_ref[...], preferred_element_type=F32, precision=lax.Precision.HIGHEST) + bp_ref[...]
    k = y[:, 0:KQ]
    q = y[:, KQ:2 * KQ]
    hs_ref[...] = jnp.concatenate(
        [y[:, 2 * KQ:2 * KQ + 4], jnp.zeros((y.shape[0], 12), F32)], axis=1)
    k_ref[...] = k / (jnp.sqrt(jnp.sum(k * k, axis=1, keepdims=True)) + 1e-8)
    q_ref[...] = q / (jnp.sqrt(jnp.sum(q * q, axis=1, keepdims=True)) + 1e-8)


def _d3_call(y0, wp1, bp1):
    nsteps = N // RB
    return pl.pallas_call(
        _d3_body,
        grid=(nsteps,),
        in_specs=[
            pl.BlockSpec((RB, HIN), lambda i: (i, 0)),
            pl.BlockSpec((HIN, 516), lambda i: (0, 0)),
            pl.BlockSpec((1, 516), lambda i: (0, 0)),
        ],
        out_specs=[
            pl.BlockSpec((RB, 16), lambda i: (i, 0)),
            pl.BlockSpec((RB, KQ), lambda i: (i, 0)),
            pl.BlockSpec((RB, KQ), lambda i: (i, 0)),
        ],
        out_shape=[
            jax.ShapeDtypeStruct((N, 16), F32),
            jax.ShapeDtypeStruct((N, KQ), F32),
            jax.ShapeDtypeStruct((N, KQ), F32),
        ],
    )(y0, wp1, bp1)


def _e_body(src_ref, dst_ref, k_ref, q_ref, hs_ref, z_ref, ep_ref,
            ebs, ebd, kbuf, qbuf, hbuf, mbuf, macc, sem, sem2, sem3):
    ci = lax.axis_index("c")
    si = lax.axis_index("s")
    wid = si * NC + ci

    @pl.when(si == 0)
    def _():
        pltpu.sync_copy(z_ref, macc)

    zero16 = jnp.zeros((16,), F32)
    for e in range(16):
        mbuf[e, :] = zero16
    plsc.subcore_barrier()

    base_w, nb = _worker_span(wid)
    rows = lax.iota(jnp.int32, 16)

    def batch_body(bi, carry):
        base = base_w + bi * 16
        pltpu.sync_copy(src_ref.at[pl.ds(base, 16)], ebs)
        pltpu.sync_copy(dst_ref.at[pl.ds(base, 16)], ebd)
        cp1 = pltpu.async_copy(k_ref.at[ebs], kbuf, sem)
        cp2 = pltpu.async_copy(q_ref.at[ebd], qbuf, sem2)
        cp3 = pltpu.async_copy(hs_ref.at[ebs], hbuf, sem3)
        cp1.wait()
        cp2.wait()
        cp3.wait()

        def col_body(j, acc):
            colj = jnp.full((16,), 0, jnp.int32) + j
            ck = plsc.load_gather(kbuf, [rows, colj])
            cq = plsc.load_gather(qbuf, [rows, colj])
            return acc + ck * cq

        acc = lax.fori_loop(0, KQ, col_body, jnp.zeros((16,), F32), unroll=8)
        pv = jnp.exp(acc * (1.0 / 16.0))
        h0 = plsc.load_gather(hbuf, [rows, jnp.full((16,), 0, jnp.int32)])
        h1 = plsc.load_gather(hbuf, [rows, jnp.full((16,), 1, jnp.int32)])
        plsc.store_scatter(mbuf, [rows, jnp.full((16,), 0, jnp.int32)], pv * h0)
        plsc.store_scatter(mbuf, [rows, jnp.full((16,), 1, jnp.int32)], pv * h1)
        plsc.store_scatter(mbuf, [rows, jnp.full((16,), 2, jnp.int32)], pv)
        pltpu.sync_copy(mbuf, macc.at[ebd], add=True)
        return carry

    lax.fori_loop(0, nb, batch_body, 0)
    plsc.subcore_barrier()

    @pl.when(si == 0)
    def _():
        pltpu.sync_copy(macc, ep_ref.at[ci])


def _e_call(src, dst, k1, q1, hs1, zn16):
    f = functools.partial(
        pl.kernel,
        out_type=jax.ShapeDtypeStruct((NC, N, 16), F32),
        mesh=_mesh,
        compiler_params=_SC_PARAMS,
        scratch_types=[
            pltpu.VMEM((16,), jnp.int32),
            pltpu.VMEM((16,), jnp.int32),
            pltpu.VMEM((16, KQ), F32),
            pltpu.VMEM((16, KQ), F32),
            pltpu.VMEM((16, 16), F32),
            pltpu.VMEM((16, 16), F32),
            pltpu.VMEM_SHARED((N, 16), F32),
            pltpu.SemaphoreType.DMA,
            pltpu.SemaphoreType.DMA,
            pltpu.SemaphoreType.DMA,
        ],
    )(_e_body)
    return f(src, dst, k1, q1, hs1, zn16)


def _f_body(ep_ref, hs_ref, o_ref):
    m = ep_ref[0] + ep_ref[1]
    rec = 1.0 / (m[:, 2:3] + 1e-16)
    o_ref[...] = jnp.maximum(hs_ref[:, 2:4] + m[:, 0:2] * rec, 0.0)


def _f_call(ep, hs1):
    return pl.pallas_call(
        _f_body,
        out_shape=jax.ShapeDtypeStruct((N, 2), F32),
    )(ep, hs1)


def kernel(x, edge_index, W0, Wk0, Wq0, SW0, Sb0, g0, b0,
           W1, Wk1, Wq1, SW1, Sb1, g1, b1):
    src = edge_index[0]
    dst = edge_index[1]
    zn16 = jnp.zeros((N, 16), F32)
    zn128 = jnp.zeros((N, D), F32)

    wcat = jnp.concatenate([W0, Wk0, Wq0, SW0], axis=1)
    sbpad = jnp.concatenate([jnp.zeros((3 * HIN,), F32), Sb0]).reshape(1, 4 * HIN)
    wp, bp = _a1_call(x, wcat, g0.reshape(1, D), b0.reshape(1, D), sbpad)
    h, kn, qn, s = _a2_call(x, wp, bp)

    p, den = _b_call(src, dst, kn, qn, zn16)
    gacc = _c_call(src, dst, p, h.reshape(N * NGRP, D), zn128)

    y0, cs, cq = _d_call(s, gacc, den)
    wcat1 = jnp.concatenate([Wk1, Wq1, W1, SW1], axis=1)
    sb1pad = jnp.concatenate([jnp.zeros((2 * KQ + 2,), F32), Sb1]).reshape(1, 516)
    wp1, bp1 = _d2_call(cs, cq, g1.reshape(1, HIN), b1.reshape(1, HIN),
                        wcat1, sb1pad)
    hs1, k1, q1 = _d3_call(y0, wp1, bp1)

    ep = _e_call(src, dst, k1, q1, hs1, zn16)
    return _f_call(ep, hs1)


# speedup vs baseline: 1.8013x; 1.0469x over previous
import functools
import jax
import jax.numpy as jnp
from jax import lax
from jax.experimental import pallas as pl
from jax.experimental.pallas import tpu as pltpu
from jax.experimental.pallas import tpu_sc as plsc

N = 10000
E = 160000
D = 128
HID = 256
HEADS = 4
KQ = 256
HIN = HEADS * HID
F32 = jnp.float32

NC = 2
NS = 16
NW = NC * NS

RB = 1000
EPW = E // NW
BB = 40
EPC = E // NS
BC = 16
NGRP = 8
GPC = NGRP // NC

_mesh = plsc.VectorSubcoreMesh(core_axis_name="c", subcore_axis_name="s")
_SC_PARAMS = pltpu.CompilerParams(use_tc_tiling_on_sc=False,
                                  needs_layout_passes=False)


def _a1_body(x_ref, w_ref, g_ref, b_ref, sb_ref, wp_ref, bp_ref):
    x = x_ref[...]
    mu = jnp.mean(x, axis=0, keepdims=True)
    var = jnp.mean(x * x, axis=0, keepdims=True) - mu * mu
    a = g_ref[...] * lax.rsqrt(var + 1e-5)
    c = b_ref[...] - mu * a
    w = w_ref[...]
    wp_ref[...] = w * jnp.transpose(a)
    bp_ref[...] = jnp.dot(c, w, preferred_element_type=F32, precision=lax.Precision.HIGHEST) + sb_ref[...]


def _a1_call(x, wcat, g, b, sbpad):
    return pl.pallas_call(
        _a1_body,
        out_shape=[
            jax.ShapeDtypeStruct((D, 4 * HIN), F32),
            jax.ShapeDtypeStruct((1, 4 * HIN), F32),
        ],
    )(x, wcat, g, b, sbpad)


def _a2_body(x_ref, wp_ref, bp_ref, h_ref, kn_ref, qn_ref, s_ref):
    y = jnp.dot(x_ref[...], wp_ref[...], preferred_element_type=F32, precision=lax.Precision.HIGHEST) + bp_ref[...]
    h_ref[...] = y[:, 0:HIN]
    s_ref[...] = y[:, 3 * HIN:4 * HIN]
    for lo, ref in ((HIN, kn_ref), (2 * HIN, qn_ref)):
        for hh in range(HEADS):
            ch = y[:, lo + hh * KQ: lo + (hh + 1) * KQ]
            nrm = jnp.sqrt(jnp.sum(ch * ch, axis=1, keepdims=True)) + 1e-8
            ref[:, hh * KQ:(hh + 1) * KQ] = ch / nrm


def _a2_call(x, wp, bp):
    nsteps = N // RB
    return pl.pallas_call(
        _a2_body,
        grid=(nsteps,),
        in_specs=[
            pl.BlockSpec((RB, D), lambda i: (i, 0)),
            pl.BlockSpec((D, 4 * HIN), lambda i: (0, 0)),
            pl.BlockSpec((1, 4 * HIN), lambda i: (0, 0)),
        ],
        out_specs=[
            pl.BlockSpec((RB, HIN), lambda i: (i, 0)),
            pl.BlockSpec((RB, HIN), lambda i: (i, 0)),
            pl.BlockSpec((RB, HIN), lambda i: (i, 0)),
            pl.BlockSpec((RB, HIN), lambda i: (i, 0)),
        ],
        out_shape=[jax.ShapeDtypeStruct((N, HIN), F32)] * 4,
    )(x, wp, bp)


EB_LO = 4992
EB_HI = 5008


def _worker_rows(wid):
    rowbase = wid * (EB_LO // 16) + jnp.minimum(wid, 16)
    nb = jnp.where(wid < 16, EB_HI // 16, EB_LO // 16)
    return rowbase, nb


def _b_compute(kb, qb, pbuf, pall, b, rows):
    def col_body(j, accs):
        colj = jnp.full((16,), 0, jnp.int32) + j
        new = []
        for hh in range(HEADS):
            cols = colj + (hh * KQ)
            ck = plsc.load_gather(kb, [rows, cols])
            cq = plsc.load_gather(qb, [rows, cols])
            new.append(accs[hh] + ck * cq)
        return tuple(new)

    accs = lax.fori_loop(0, KQ, col_body,
                         tuple(jnp.zeros((16,), F32) for _ in range(HEADS)),
                         unroll=4)
    for hh in range(HEADS):
        pv = jnp.exp(accs[hh] * (1.0 / 16.0))
        pall[hh, pl.ds(b * 16, 16)] = pv
        plsc.store_scatter(pbuf, [rows, jnp.full((16,), hh, jnp.int32)], pv)


def _b_body(src_ref, dst_ref, kn_ref, qn_ref, z_ref, p_ref, den_ref,
            ebs2, ebd2, kb0, qb0, kb1, qb1, pbuf0, pbuf1, pall, dacc,
            gs0, gs1, gs2, gs3, ss0, ss1):
    ci = lax.axis_index("c")
    si = lax.axis_index("s")
    wid = si * NC + ci

    @pl.when(si == 0)
    def _():
        pltpu.sync_copy(z_ref, dacc)

    zero16 = jnp.zeros((16,), F32)
    for e in range(16):
        pbuf0[e, :] = zero16
        pbuf1[e, :] = zero16

    rowbase, nb = _worker_rows(wid)
    pltpu.sync_copy(src_ref.at[pl.ds(rowbase, EB_LO // 16), :],
                    ebs2.at[pl.ds(0, EB_LO // 16), :])
    pltpu.sync_copy(dst_ref.at[pl.ds(rowbase, EB_LO // 16), :],
                    ebd2.at[pl.ds(0, EB_LO // 16), :])

    @pl.when(wid < 16)
    def _():
        pltpu.sync_copy(src_ref.at[pl.ds(rowbase + EB_LO // 16, 1), :],
                        ebs2.at[pl.ds(EB_LO // 16, 1), :])
        pltpu.sync_copy(dst_ref.at[pl.ds(rowbase + EB_LO // 16, 1), :],
                        ebd2.at[pl.ds(EB_LO // 16, 1), :])

    plsc.subcore_barrier()
    rows = lax.iota(jnp.int32, 16)

    def pair_body(k, carry):
        b0 = 2 * k
        b1 = 2 * k + 1
        cka = pltpu.async_copy(kn_ref.at[ebs2.at[b0]], kb0, gs0)
        cqa = pltpu.async_copy(qn_ref.at[ebd2.at[b0]], qb0, gs1)
        ckb = pltpu.async_copy(kn_ref.at[ebs2.at[b1]], kb1, gs2)
        cqb = pltpu.async_copy(qn_ref.at[ebd2.at[b1]], qb1, gs3)
        cka.wait()
        cqa.wait()
        _b_compute(kb0, qb0, pbuf0, pall, b0, rows)
        sca = pltpu.async_copy(pbuf0, dacc.at[ebd2.at[b0]], ss0, add=True)
        ckb.wait()
        cqb.wait()
        _b_compute(kb1, qb1, pbuf1, pall, b1, rows)
        scb = pltpu.async_copy(pbuf1, dacc.at[ebd2.at[b1]], ss1, add=True)
        sca.wait()
        scb.wait()
        return carry

    lax.fori_loop(0, nb // 2, pair_body, 0)

    @pl.when(nb % 2 == 1)
    def _():
        bt = nb - 1
        pltpu.async_copy(kn_ref.at[ebs2.at[bt]], kb0, gs0).wait()
        pltpu.async_copy(qn_ref.at[ebd2.at[bt]], qb0, gs1).wait()
        _b_compute(kb0, qb0, pbuf0, pall, bt, rows)
        pltpu.sync_copy(pbuf0, dacc.at[ebd2.at[bt]], add=True)

    base_w = rowbase * 16

    @pl.when(wid < 16)
    def _():
        for hh in range(HEADS):
            pltpu.sync_copy(pall.at[hh, pl.ds(0, EB_HI)],
                            p_ref.at[pl.ds(hh * E + base_w, EB_HI)])

    @pl.when(wid >= 16)
    def _():
        for hh in range(HEADS):
            pltpu.sync_copy(pall.at[hh, pl.ds(0, EB_LO)],
                            p_ref.at[pl.ds(hh * E + base_w, EB_LO)])

    plsc.subcore_barrier()

    @pl.when(si == 0)
    def _():
        pltpu.sync_copy(dacc, den_ref.at[ci])


def _b_call(src, dst, kn, qn, zn16):
    f = functools.partial(
        pl.kernel,
        out_type=[
            jax.ShapeDtypeStruct((HEADS * E,), F32),
            jax.ShapeDtypeStruct((NC, N, 16), F32),
        ],
        mesh=_mesh,
        compiler_params=_SC_PARAMS,
        scratch_types=[
            pltpu.VMEM((EB_HI // 16, 16), jnp.int32),
            pltpu.VMEM((EB_HI // 16, 16), jnp.int32),
            pltpu.VMEM((16, HEADS * KQ), F32),
            pltpu.VMEM((16, HEADS * KQ), F32),
            pltpu.VMEM((16, HEADS * KQ), F32),
            pltpu.VMEM((16, HEADS * KQ), F32),
            pltpu.VMEM((16, 16), F32),
            pltpu.VMEM((16, 16), F32),
            pltpu.VMEM((HEADS, EB_HI), F32),
            pltpu.VMEM_SHARED((N, 16), F32),
            pltpu.SemaphoreType.DMA,
            pltpu.SemaphoreType.DMA,
            pltpu.SemaphoreType.DMA,
            pltpu.SemaphoreType.DMA,
            pltpu.SemaphoreType.DMA,
            pltpu.SemaphoreType.DMA,
        ],
    )(_b_body)
    return f(src.reshape(E // 16, 16), dst.reshape(E // 16, 16), kn, qn, zn16)


CB = 80
CROWS = E // CB // NS
CCH = 25


def _c_scale(hb, pb2, bi, rows_list):
    pvs = [pb2[bi, pl.ds(r * 16, 16)] for r in range(5)]

    def col_body(c, c2):
        cc = jnp.full((16,), 0, jnp.int32) + c
        for r in range(5):
            v = plsc.load_gather(hb, [rows_list[r], cc])
            plsc.store_scatter(hb, [rows_list[r], cc], v * pvs[r])
        return c2

    lax.fori_loop(0, D, col_body, 0, unroll=4)


def _c_body(src_ref, dst_ref, p_ref, h8_ref, z_ref, g_out,
            ebd2, pb2, idx82, hb0, hb1, gacc, sem0, sem1, ssem0, ssem1):
    ci = lax.axis_index("c")
    si = lax.axis_index("s")
    rb = si * CROWS
    rows_list = [lax.iota(jnp.int32, 16) + r * 16 for r in range(5)]
    for gi in range(GPC):
        g = ci * GPC + gi
        head = g // 2

        @pl.when(si == 0)
        def _():
            pltpu.sync_copy(z_ref, gacc)

        plsc.subcore_barrier()

        def chunk_body(ch, c0):
            crb = rb + ch * CCH
            pltpu.sync_copy(src_ref.at[pl.ds(crb, CCH), :], idx82)
            pltpu.sync_copy(dst_ref.at[pl.ds(crb, CCH), :], ebd2)
            pltpu.sync_copy(p_ref.at[pl.ds(head * (NS * CROWS) + crb, CCH), :],
                            pb2)

            def idx_body(r, c2):
                for cc in range(5):
                    sl = pl.ds(cc * 16, 16)
                    idx82[r, sl] = idx82[r, sl] * 8 + g
                return c2

            lax.fori_loop(0, CCH, idx_body, 0)

            def pair_body(k, c2):
                b0 = 2 * k
                b1 = 2 * k + 1
                cpa = pltpu.async_copy(h8_ref.at[idx82.at[b0]], hb0, sem0)
                cpb = pltpu.async_copy(h8_ref.at[idx82.at[b1]], hb1, sem1)
                cpa.wait()
                _c_scale(hb0, pb2, b0, rows_list)
                sca = pltpu.async_copy(hb0, gacc.at[ebd2.at[b0]], ssem0,
                                       add=True)
                cpb.wait()
                _c_scale(hb1, pb2, b1, rows_list)
                scb = pltpu.async_copy(hb1, gacc.at[ebd2.at[b1]], ssem1,
                                       add=True)
                sca.wait()
                scb.wait()
                return c2

            lax.fori_loop(0, CCH // 2, pair_body, 0)
            bt = CCH - 1
            pltpu.async_copy(h8_ref.at[idx82.at[bt]], hb0, sem0).wait()
            _c_scale(hb0, pb2, bt, rows_list)
            pltpu.sync_copy(hb0, gacc.at[ebd2.at[bt]], add=True)
            return c0

        lax.fori_loop(0, CROWS // CCH, chunk_body, 0)
        plsc.subcore_barrier()

        @pl.when(si == 0)
        def _():
            pltpu.sync_copy(gacc, g_out.at[g])

        plsc.subcore_barrier()


def _c_call(src, dst, p, h8, zn128):
    f = functools.partial(
        pl.kernel,
        out_type=jax.ShapeDtypeStruct((NGRP, N, D), F32),
        mesh=_mesh,
        compiler_params=_SC_PARAMS,
        scratch_types=[
            pltpu.VMEM((CCH, CB), jnp.int32),
            pltpu.VMEM((CCH, CB), F32),
            pltpu.VMEM((CCH, CB), jnp.int32),
            pltpu.VMEM((CB, D), F32),
            pltpu.VMEM((CB, D), F32),
            pltpu.VMEM_SHARED((N, D), F32),
            pltpu.SemaphoreType.DMA,
            pltpu.SemaphoreType.DMA,
            pltpu.SemaphoreType.DMA,
            pltpu.SemaphoreType.DMA,
        ],
    )(_c_body)
    return f(src.reshape(E // CB, CB), dst.reshape(E // CB, CB),
             p.reshape(HEADS * (E // CB), CB), h8, zn128)


def _d_body(s_ref, g_ref, den_ref, y_ref, cs_ref, cq_ref):
    g = pl.program_id(0)
    i = pl.program_id(1)
    head = g // 2
    den = den_ref[0] + den_ref[1]
    onehot = (lax.broadcasted_iota(jnp.int32, (1, 16), 1) == head).astype(F32)
    dh = jnp.sum(den * onehot, axis=1, keepdims=True)
    rec = 1.0 / (dh + 1e-16)
    y = jnp.maximum(s_ref[...] + g_ref[0] * rec, 0.0)
    y_ref[...] = y

    @pl.when(i == 0)
    def _():
        cs_ref[...] = jnp.zeros_like(cs_ref)
        cq_ref[...] = jnp.zeros_like(cq_ref)

    cs_ref[...] += jnp.sum(y, axis=0, keepdims=True)
    cq_ref[...] += jnp.sum(y * y, axis=0, keepdims=True)


def _d_call(s, gacc, den):
    nsteps = N // RB
    return pl.pallas_call(
        _d_body,
        grid=(NGRP, nsteps),
        in_specs=[
            pl.BlockSpec((RB, D), lambda g, i: (i, g)),
            pl.BlockSpec((1, RB, D), lambda g, i: (g, i, 0)),
            pl.BlockSpec((NC, RB, 16), lambda g, i: (0, i, 0)),
        ],
        out_specs=[
            pl.BlockSpec((RB, D), lambda g, i: (i, g)),
            pl.BlockSpec((1, D), lambda g, i: (0, g)),
            pl.BlockSpec((1, D), lambda g, i: (0, g)),
        ],
        out_shape=[
            jax.ShapeDtypeStruct((N, HIN), F32),
            jax.ShapeDtypeStruct((1, HIN), F32),
            jax.ShapeDtypeStruct((1, HIN), F32),
        ],
    )(s, gacc, den)


def _d2_body(cs_ref, cq_ref, g_ref, b_ref, w_ref, sb_ref, wp_ref, bp_ref):
    mu = cs_ref[...] * (1.0 / N)
    var = cq_ref[...] * (1.0 / N) - mu * mu
    a = g_ref[...] * lax.rsqrt(var + 1e-5)
    c = b_ref[...] - mu * a
    w = w_ref[...]
    wp_ref[...] = w * jnp.transpose(a)
    bp_ref[...] = jnp.dot(c, w, preferred_element_type=F32, precision=lax.Precision.HIGHEST) + sb_ref[...]


def _d2_call(cs, cq, g, b, wcat1, sb1pad):
    return pl.pallas_call(
        _d2_body,
        out_shape=[
            jax.ShapeDtypeStruct((HIN, 516), F32),
            jax.ShapeDtypeStruct((1, 516), F32),
        ],
    )(cs, cq, g, b, wcat1, sb1pad)


def _d3_body(y_ref, wp_ref, bp_ref, hs_ref, k_ref, q_ref):
    y = jnp.dot(y_ref[...], wp_ref[...], preferred_element_type=F32, precision=lax.Precision.HIGHEST) + bp_ref[...]
    k = y[:, 0:KQ]
    q = y[:, KQ:2 * KQ]
    hs_ref[...] = jnp.concatenate(
        [y[:, 2 * KQ:2 * KQ + 4], jnp.zeros((y.shape[0], 12), F32)], axis=1)
    k_ref[...] = k / (jnp.sqrt(jnp.sum(k * k, axis=1, keepdims=True)) + 1e-8)
    q_ref[...] = q / (jnp.sqrt(jnp.sum(q * q, axis=1, keepdims=True)) + 1e-8)


def _d3_call(y0, wp1, bp1):
    nsteps = N // RB
    return pl.pallas_call(
        _d3_body,
        grid=(nsteps,),
        in_specs=[
            pl.BlockSpec((RB, HIN), lambda i: (i, 0)),
            pl.BlockSpec((HIN, 516), lambda i: (0, 0)),
            pl.BlockSpec((1, 516), lambda i: (0, 0)),
        ],
        out_specs=[
            pl.BlockSpec((RB, 16), lambda i: (i, 0)),
            pl.BlockSpec((RB, KQ), lambda i: (i, 0)),
            pl.BlockSpec((RB, KQ), lambda i: (i, 0)),
        ],
        out_shape=[
            jax.ShapeDtypeStruct((N, 16), F32),
            jax.ShapeDtypeStruct((N, KQ), F32),
            jax.ShapeDtypeStruct((N, KQ), F32),
        ],
    )(y0, wp1, bp1)


def _e_compute(kb, qb, hsb, mbuf, rows):
    def col_body(j, acc):
        colj = jnp.full((16,), 0, jnp.int32) + j
        ck = plsc.load_gather(kb, [rows, colj])
        cq = plsc.load_gather(qb, [rows, colj])
        return acc + ck * cq

    acc = lax.fori_loop(0, KQ, col_body, jnp.zeros((16,), F32), unroll=8)
    pv = jnp.exp(acc * (1.0 / 16.0))
    h0 = plsc.load_gather(hsb, [rows, jnp.full((16,), 0, jnp.int32)])
    h1 = plsc.load_gather(hsb, [rows, jnp.full((16,), 1, jnp.int32)])
    plsc.store_scatter(mbuf, [rows, jnp.full((16,), 0, jnp.int32)], pv * h0)
    plsc.store_scatter(mbuf, [rows, jnp.full((16,), 1, jnp.int32)], pv * h1)
    plsc.store_scatter(mbuf, [rows, jnp.full((16,), 2, jnp.int32)], pv)


def _e_body(src_ref, dst_ref, k_ref, q_ref, hs_ref, z_ref, ep_ref,
            ebs2, ebd2, kb0, qb0, hsb0, kb1, qb1, hsb1, mbuf0, mbuf1, macc,
            gs0, gs1, gs2, gs3, gs4, gs5, ss0, ss1):
    ci = lax.axis_index("c")
    si = lax.axis_index("s")
    wid = si * NC + ci

    @pl.when(si == 0)
    def _():
        pltpu.sync_copy(z_ref, macc)

    zero16 = jnp.zeros((16,), F32)
    for e in range(16):
        mbuf0[e, :] = zero16
        mbuf1[e, :] = zero16

    rowbase, nb = _worker_rows(wid)
    pltpu.sync_copy(src_ref.at[pl.ds(rowbase, EB_LO // 16), :],
                    ebs2.at[pl.ds(0, EB_LO // 16), :])
    pltpu.sync_copy(dst_ref.at[pl.ds(rowbase, EB_LO // 16), :],
                    ebd2.at[pl.ds(0, EB_LO // 16), :])

    @pl.when(wid < 16)
    def _():
        pltpu.sync_copy(src_ref.at[pl.ds(rowbase + EB_LO // 16, 1), :],
                        ebs2.at[pl.ds(EB_LO // 16, 1), :])
        pltpu.sync_copy(dst_ref.at[pl.ds(rowbase + EB_LO // 16, 1), :],
                        ebd2.at[pl.ds(EB_LO // 16, 1), :])

    plsc.subcore_barrier()
    rows = lax.iota(jnp.int32, 16)

    def pair_body(k, carry):
        b0 = 2 * k
        b1 = 2 * k + 1
        cka = pltpu.async_copy(k_ref.at[ebs2.at[b0]], kb0, gs0)
        cqa = pltpu.async_copy(q_ref.at[ebd2.at[b0]], qb0, gs1)
        cha = pltpu.async_copy(hs_ref.at[ebs2.at[b0]], hsb0, gs2)
        ckb = pltpu.async_copy(k_ref.at[ebs2.at[b1]], kb1, gs3)
        cqb = pltpu.async_copy(q_ref.at[ebd2.at[b1]], qb1, gs4)
        chb = pltpu.async_copy(hs_ref.at[ebs2.at[b1]], hsb1, gs5)
        cka.wait()
        cqa.wait()
        cha.wait()
        _e_compute(kb0, qb0, hsb0, mbuf0, rows)
        sca = pltpu.async_copy(mbuf0, macc.at[ebd2.at[b0]], ss0, add=True)
        ckb.wait()
        cqb.wait()
        chb.wait()
        _e_compute(kb1, qb1, hsb1, mbuf1, rows)
        scb = pltpu.async_copy(mbuf1, macc.at[ebd2.at[b1]], ss1, add=True)
        sca.wait()
        scb.wait()
        return carry

    lax.fori_loop(0, nb // 2, pair_body, 0)

    @pl.when(nb % 2 == 1)
    def _():
        bt = nb - 1
        pltpu.async_copy(k_ref.at[ebs2.at[bt]], kb0, gs0).wait()
        pltpu.async_copy(q_ref.at[ebd2.at[bt]], qb0, gs1).wait()
        pltpu.async_copy(hs_ref.at[ebs2.at[bt]], hsb0, gs2).wait()
        _e_compute(kb0, qb0, hsb0, mbuf0, rows)
        pltpu.sync_copy(mbuf0, macc.at[ebd2.at[bt]], add=True)

    plsc.subcore_barrier()

    @pl.when(si == 0)
    def _():
        pltpu.sync_copy(macc, ep_ref.at[ci])


def _e_call(src, dst, k1, q1, hs1, zn16):
    f = functools.partial(
        pl.kernel,
        out_type=jax.ShapeDtypeStruct((NC, N, 16), F32),
        mesh=_mesh,
        compiler_params=_SC_PARAMS,
        scratch_types=[
            pltpu.VMEM((EB_HI // 16, 16), jnp.int32),
            pltpu.VMEM((EB_HI // 16, 16), jnp.int32),
            pltpu.VMEM((16, KQ), F32),
            pltpu.VMEM((16, KQ), F32),
            pltpu.VMEM((16, 16), F32),
            pltpu.VMEM((16, KQ), F32),
            pltpu.VMEM((16, KQ), F32),
            pltpu.VMEM((16, 16), F32),
            pltpu.VMEM((16, 16), F32),
            pltpu.VMEM((16, 16), F32),
            pltpu.VMEM_SHARED((N, 16), F32),
            pltpu.SemaphoreType.DMA,
            pltpu.SemaphoreType.DMA,
            pltpu.SemaphoreType.DMA,
            pltpu.SemaphoreType.DMA,
            pltpu.SemaphoreType.DMA,
            pltpu.SemaphoreType.DMA,
            pltpu.SemaphoreType.DMA,
            pltpu.SemaphoreType.DMA,
        ],
    )(_e_body)
    return f(src.reshape(E // 16, 16), dst.reshape(E // 16, 16), k1, q1, hs1,
             zn16)


def _f_body(ep_ref, hs_ref, o_ref):
    m = ep_ref[0] + ep_ref[1]
    rec = 1.0 / (m[:, 2:3] + 1e-16)
    o_ref[...] = jnp.maximum(hs_ref[:, 2:4] + m[:, 0:2] * rec, 0.0)


def _f_call(ep, hs1):
    return pl.pallas_call(
        _f_body,
        out_shape=jax.ShapeDtypeStruct((N, 2), F32),
    )(ep, hs1)


def kernel(x, edge_index, W0, Wk0, Wq0, SW0, Sb0, g0, b0,
           W1, Wk1, Wq1, SW1, Sb1, g1, b1):
    src = edge_index[0]
    dst = edge_index[1]
    zn16 = jnp.zeros((N, 16), F32)
    zn128 = jnp.zeros((N, D), F32)

    wcat = jnp.concatenate([W0, Wk0, Wq0, SW0], axis=1)
    sbpad = jnp.concatenate([jnp.zeros((3 * HIN,), F32), Sb0]).reshape(1, 4 * HIN)
    wp, bp = _a1_call(x, wcat, g0.reshape(1, D), b0.reshape(1, D), sbpad)
    h, kn, qn, s = _a2_call(x, wp, bp)

    p, den = _b_call(src, dst, kn, qn, zn16)
    gacc = _c_call(src, dst, p, h.reshape(N * NGRP, D), zn128)

    y0, cs, cq = _d_call(s, gacc, den)
    wcat1 = jnp.concatenate([Wk1, Wq1, W1, SW1], axis=1)
    sb1pad = jnp.concatenate([jnp.zeros((2 * KQ + 2,), F32), Sb1]).reshape(1, 516)
    wp1, bp1 = _d2_call(cs, cq, g1.reshape(1, HIN), b1.reshape(1, HIN),
                        wcat1, sb1pad)
    hs1, k1, q1 = _d3_call(y0, wp1, bp1)

    ep = _e_call(src, dst, k1, q1, hs1, zn16)
    return _f_call(ep, hs1)


# speedup vs baseline: 2.3904x; 1.3271x over previous
import functools
import jax
import jax.numpy as jnp
from jax import lax
from jax.experimental import pallas as pl
from jax.experimental.pallas import tpu as pltpu
from jax.experimental.pallas import tpu_sc as plsc

N = 10000
E = 160000
D = 128
HID = 256
HEADS = 4
KQ = 256
HIN = HEADS * HID
F32 = jnp.float32
BF16 = jnp.bfloat16
KQI = KQ // 2
DI = D // 2

NC = 2
NS = 16
NW = NC * NS

RB = 1000
EPW = E // NW
BB = 40
EPC = E // NS
BC = 16
NGRP = 8
GPC = NGRP // NC

_mesh = plsc.VectorSubcoreMesh(core_axis_name="c", subcore_axis_name="s")
_SC_PARAMS = pltpu.CompilerParams(use_tc_tiling_on_sc=False,
                                  needs_layout_passes=False)


def _a1_body(x_ref, w_ref, g_ref, b_ref, sb_ref, wp_ref, bp_ref):
    x = x_ref[...]
    mu = jnp.mean(x, axis=0, keepdims=True)
    var = jnp.mean(x * x, axis=0, keepdims=True) - mu * mu
    a = g_ref[...] * lax.rsqrt(var + 1e-5)
    c = b_ref[...] - mu * a
    w = w_ref[...]
    wp_ref[...] = w * jnp.transpose(a)
    bp_ref[...] = jnp.dot(c, w, preferred_element_type=F32, precision=lax.Precision.HIGHEST) + sb_ref[...]


def _a1_call(x, wcat, g, b, sbpad):
    return pl.pallas_call(
        _a1_body,
        out_shape=[
            jax.ShapeDtypeStruct((D, 4 * HIN), F32),
            jax.ShapeDtypeStruct((1, 4 * HIN), F32),
        ],
    )(x, wcat, g, b, sbpad)


def _a2_body(x_ref, wp_ref, bp_ref, h_ref, kn_ref, qn_ref, s_ref):
    y = jnp.dot(x_ref[...], wp_ref[...], preferred_element_type=F32, precision=lax.Precision.HIGHEST) + bp_ref[...]
    h_ref[...] = y[:, 0:HIN].astype(BF16)
    s_ref[...] = y[:, 3 * HIN:4 * HIN]
    for lo, ref in ((HIN, kn_ref), (2 * HIN, qn_ref)):
        for hh in range(HEADS):
            ch = y[:, lo + hh * KQ: lo + (hh + 1) * KQ]
            nrm = jnp.sqrt(jnp.sum(ch * ch, axis=1, keepdims=True)) + 1e-8
            ref[:, hh * KQ:(hh + 1) * KQ] = (ch / nrm).astype(BF16)


def _a2_call(x, wp, bp):
    nsteps = N // RB
    return pl.pallas_call(
        _a2_body,
        grid=(nsteps,),
        in_specs=[
            pl.BlockSpec((RB, D), lambda i: (i, 0)),
            pl.BlockSpec((D, 4 * HIN), lambda i: (0, 0)),
            pl.BlockSpec((1, 4 * HIN), lambda i: (0, 0)),
        ],
        out_specs=[
            pl.BlockSpec((RB, HIN), lambda i: (i, 0)),
            pl.BlockSpec((RB, HIN), lambda i: (i, 0)),
            pl.BlockSpec((RB, HIN), lambda i: (i, 0)),
            pl.BlockSpec((RB, HIN), lambda i: (i, 0)),
        ],
        out_shape=[
            jax.ShapeDtypeStruct((N, HIN), BF16),
            jax.ShapeDtypeStruct((N, HIN), BF16),
            jax.ShapeDtypeStruct((N, HIN), BF16),
            jax.ShapeDtypeStruct((N, HIN), F32),
        ],
    )(x, wp, bp)


EB_LO = 4992
EB_HI = 5008


def _worker_rows(wid):
    rowbase = wid * (EB_LO // 16) + jnp.minimum(wid, 16)
    nb = jnp.where(wid < 16, EB_HI // 16, EB_LO // 16)
    return rowbase, nb


def _bf_pair(v, sh16, mhi):
    lo = lax.bitcast_convert_type(lax.shift_left(v, sh16), F32)
    hi = lax.bitcast_convert_type(v & mhi, F32)
    return lo, hi


def _b_compute(kb, qb, pbuf, pall, b, rows):
    sh16 = jnp.full((16,), 16, jnp.int32)
    mhi = jnp.full((16,), -65536, jnp.int32)

    def col_body(j, accs):
        colj = jnp.full((16,), 0, jnp.int32) + j
        new = []
        for hh in range(HEADS):
            cols = colj + (hh * KQI)
            ck = plsc.load_gather(kb, [rows, cols])
            cq = plsc.load_gather(qb, [rows, cols])
            ke, ko = _bf_pair(ck, sh16, mhi)
            qe, qo = _bf_pair(cq, sh16, mhi)
            new.append(accs[hh] + ke * qe + ko * qo)
        return tuple(new)

    accs = lax.fori_loop(0, KQI, col_body,
                         tuple(jnp.zeros((16,), F32) for _ in range(HEADS)),
                         unroll=4)
    for hh in range(HEADS):
        pv = jnp.exp(accs[hh] * (1.0 / 16.0))
        pall[hh, pl.ds(b * 16, 16)] = pv
        plsc.store_scatter(pbuf, [rows, jnp.full((16,), hh, jnp.int32)], pv)


def _b_body(src_ref, dst_ref, kn_ref, qn_ref, z_ref, p_ref, den_ref,
            ebs2, ebd2, kb0, qb0, kb1, qb1, pbuf0, pbuf1, pall, dacc,
            gs0, gs1, gs2, gs3, ss0, ss1):
    ci = lax.axis_index("c")
    si = lax.axis_index("s")
    wid = si * NC + ci

    @pl.when(si == 0)
    def _():
        pltpu.sync_copy(z_ref, dacc)

    zero16 = jnp.zeros((16,), F32)
    for e in range(16):
        pbuf0[e, :] = zero16
        pbuf1[e, :] = zero16

    rowbase, nb = _worker_rows(wid)
    pltpu.sync_copy(src_ref.at[pl.ds(rowbase, EB_LO // 16), :],
                    ebs2.at[pl.ds(0, EB_LO // 16), :])
    pltpu.sync_copy(dst_ref.at[pl.ds(rowbase, EB_LO // 16), :],
                    ebd2.at[pl.ds(0, EB_LO // 16), :])

    @pl.when(wid < 16)
    def _():
        pltpu.sync_copy(src_ref.at[pl.ds(rowbase + EB_LO // 16, 1), :],
                        ebs2.at[pl.ds(EB_LO // 16, 1), :])
        pltpu.sync_copy(dst_ref.at[pl.ds(rowbase + EB_LO // 16, 1), :],
                        ebd2.at[pl.ds(EB_LO // 16, 1), :])

    plsc.subcore_barrier()
    rows = lax.iota(jnp.int32, 16)

    def pair_body(k, carry):
        b0 = 2 * k
        b1 = 2 * k + 1
        cka = pltpu.async_copy(kn_ref.at[ebs2.at[b0]], kb0, gs0)
        cqa = pltpu.async_copy(qn_ref.at[ebd2.at[b0]], qb0, gs1)
        ckb = pltpu.async_copy(kn_ref.at[ebs2.at[b1]], kb1, gs2)
        cqb = pltpu.async_copy(qn_ref.at[ebd2.at[b1]], qb1, gs3)
        cka.wait()
        cqa.wait()
        _b_compute(kb0, qb0, pbuf0, pall, b0, rows)
        sca = pltpu.async_copy(pbuf0, dacc.at[ebd2.at[b0]], ss0, add=True)
        ckb.wait()
        cqb.wait()
        _b_compute(kb1, qb1, pbuf1, pall, b1, rows)
        scb = pltpu.async_copy(pbuf1, dacc.at[ebd2.at[b1]], ss1, add=True)
        sca.wait()
        scb.wait()
        return carry

    lax.fori_loop(0, nb // 2, pair_body, 0)

    @pl.when(nb % 2 == 1)
    def _():
        bt = nb - 1
        pltpu.async_copy(kn_ref.at[ebs2.at[bt]], kb0, gs0).wait()
        pltpu.async_copy(qn_ref.at[ebd2.at[bt]], qb0, gs1).wait()
        _b_compute(kb0, qb0, pbuf0, pall, bt, rows)
        pltpu.sync_copy(pbuf0, dacc.at[ebd2.at[bt]], add=True)

    base_w = rowbase * 16

    @pl.when(wid < 16)
    def _():
        for hh in range(HEADS):
            pltpu.sync_copy(pall.at[hh, pl.ds(0, EB_HI)],
                            p_ref.at[pl.ds(hh * E + base_w, EB_HI)])

    @pl.when(wid >= 16)
    def _():
        for hh in range(HEADS):
            pltpu.sync_copy(pall.at[hh, pl.ds(0, EB_LO)],
                            p_ref.at[pl.ds(hh * E + base_w, EB_LO)])

    plsc.subcore_barrier()

    @pl.when(si == 0)
    def _():
        pltpu.sync_copy(dacc, den_ref.at[ci])


def _b_call(src, dst, kn, qn, zn16):
    f = functools.partial(
        pl.kernel,
        out_type=[
            jax.ShapeDtypeStruct((HEADS * E,), F32),
            jax.ShapeDtypeStruct((NC, N, 16), F32),
        ],
        mesh=_mesh,
        compiler_params=_SC_PARAMS,
        scratch_types=[
            pltpu.VMEM((EB_HI // 16, 16), jnp.int32),
            pltpu.VMEM((EB_HI // 16, 16), jnp.int32),
            pltpu.VMEM((16, HEADS * KQI), jnp.int32),
            pltpu.VMEM((16, HEADS * KQI), jnp.int32),
            pltpu.VMEM((16, HEADS * KQI), jnp.int32),
            pltpu.VMEM((16, HEADS * KQI), jnp.int32),
            pltpu.VMEM((16, 16), F32),
            pltpu.VMEM((16, 16), F32),
            pltpu.VMEM((HEADS, EB_HI), F32),
            pltpu.VMEM_SHARED((N, 16), F32),
            pltpu.SemaphoreType.DMA,
            pltpu.SemaphoreType.DMA,
            pltpu.SemaphoreType.DMA,
            pltpu.SemaphoreType.DMA,
            pltpu.SemaphoreType.DMA,
            pltpu.SemaphoreType.DMA,
        ],
    )(_b_body)
    return f(src.reshape(E // 16, 16), dst.reshape(E // 16, 16), kn, qn, zn16)


CB = 80
CROWS = E // CB // NS
CCH = 25


def _c_scale(hb, ob, pb2, bi, rows_list):
    pvs = [pb2[bi, pl.ds(r * 16, 16)] for r in range(5)]
    sh16 = jnp.full((16,), 16, jnp.int32)
    mhi = jnp.full((16,), -65536, jnp.int32)
    one = jnp.full((16,), 1, jnp.int32)

    def col_body(c, c2):
        cc = jnp.full((16,), 0, jnp.int32) + c
        ce = cc + cc
        for r in range(5):
            v = plsc.load_gather(hb, [rows_list[r], cc])
            lo, hi = _bf_pair(v, sh16, mhi)
            plsc.store_scatter(ob, [rows_list[r], ce], lo * pvs[r])
            plsc.store_scatter(ob, [rows_list[r], ce + one], hi * pvs[r])
        return c2

    lax.fori_loop(0, DI, col_body, 0, unroll=4)


def _c_body(src_ref, dst_ref, p_ref, h8_ref, z_ref, g_out,
            ebd2, pb2, idx82, hb0, hb1, ob0, ob1, gacc,
            sem0, sem1, ssem0, ssem1):
    ci = lax.axis_index("c")
    si = lax.axis_index("s")
    rb = si * CROWS
    rows_list = [lax.iota(jnp.int32, 16) + r * 16 for r in range(5)]
    for gi in range(GPC):
        g = ci * GPC + gi
        head = g // 2

        @pl.when(si == 0)
        def _():
            pltpu.sync_copy(z_ref, gacc)

        plsc.subcore_barrier()

        def chunk_body(ch, c0):
            crb = rb + ch * CCH
            pltpu.sync_copy(src_ref.at[pl.ds(crb, CCH), :], idx82)
            pltpu.sync_copy(dst_ref.at[pl.ds(crb, CCH), :], ebd2)
            pltpu.sync_copy(p_ref.at[pl.ds(head * (NS * CROWS) + crb, CCH), :],
                            pb2)

            def idx_body(r, c2):
                for cc in range(5):
                    sl = pl.ds(cc * 16, 16)
                    idx82[r, sl] = idx82[r, sl] * 8 + g
                return c2

            lax.fori_loop(0, CCH, idx_body, 0)

            def pair_body(k, c2):
                b0 = 2 * k
                b1 = 2 * k + 1
                cpa = pltpu.async_copy(h8_ref.at[idx82.at[b0]], hb0, sem0)
                cpb = pltpu.async_copy(h8_ref.at[idx82.at[b1]], hb1, sem1)
                cpa.wait()
                _c_scale(hb0, ob0, pb2, b0, rows_list)
                sca = pltpu.async_copy(ob0, gacc.at[ebd2.at[b0]], ssem0,
                                       add=True)
                cpb.wait()
                _c_scale(hb1, ob1, pb2, b1, rows_list)
                scb = pltpu.async_copy(ob1, gacc.at[ebd2.at[b1]], ssem1,
                                       add=True)
                sca.wait()
                scb.wait()
                return c2

            lax.fori_loop(0, CCH // 2, pair_body, 0)
            bt = CCH - 1
            pltpu.async_copy(h8_ref.at[idx82.at[bt]], hb0, sem0).wait()
            _c_scale(hb0, ob0, pb2, bt, rows_list)
            pltpu.sync_copy(ob0, gacc.at[ebd2.at[bt]], add=True)
            return c0

        lax.fori_loop(0, CROWS // CCH, chunk_body, 0)
        plsc.subcore_barrier()

        @pl.when(si == 0)
        def _():
            pltpu.sync_copy(gacc, g_out.at[g])

        plsc.subcore_barrier()


def _c_call(src, dst, p, h8, zn128):
    f = functools.partial(
        pl.kernel,
        out_type=jax.ShapeDtypeStruct((NGRP, N, D), F32),
        mesh=_mesh,
        compiler_params=_SC_PARAMS,
        scratch_types=[
            pltpu.VMEM((CCH, CB), jnp.int32),
            pltpu.VMEM((CCH, CB), F32),
            pltpu.VMEM((CCH, CB), jnp.int32),
            pltpu.VMEM((CB, DI), jnp.int32),
            pltpu.VMEM((CB, DI), jnp.int32),
            pltpu.VMEM((CB, D), F32),
            pltpu.VMEM((CB, D), F32),
            pltpu.VMEM_SHARED((N, D), F32),
            pltpu.SemaphoreType.DMA,
            pltpu.SemaphoreType.DMA,
            pltpu.SemaphoreType.DMA,
            pltpu.SemaphoreType.DMA,
        ],
    )(_c_body)
    return f(src.reshape(E // CB, CB), dst.reshape(E // CB, CB),
             p.reshape(HEADS * (E // CB), CB), h8, zn128)


def _d_body(s_ref, g_ref, den_ref, y_ref, cs_ref, cq_ref):
    g = pl.program_id(0)
    i = pl.program_id(1)
    head = g // 2
    den = den_ref[0] + den_ref[1]
    onehot = (lax.broadcasted_iota(jnp.int32, (1, 16), 1) == head).astype(F32)
    dh = jnp.sum(den * onehot, axis=1, keepdims=True)
    rec = 1.0 / (dh + 1e-16)
    y = jnp.maximum(s_ref[...] + g_ref[0] * rec, 0.0)
    y_ref[...] = y

    @pl.when(i == 0)
    def _():
        cs_ref[...] = jnp.zeros_like(cs_ref)
        cq_ref[...] = jnp.zeros_like(cq_ref)

    cs_ref[...] += jnp.sum(y, axis=0, keepdims=True)
    cq_ref[...] += jnp.sum(y * y, axis=0, keepdims=True)


def _d_call(s, gacc, den):
    nsteps = N // RB
    return pl.pallas_call(
        _d_body,
        grid=(NGRP, nsteps),
        in_specs=[
            pl.BlockSpec((RB, D), lambda g, i: (i, g)),
            pl.BlockSpec((1, RB, D), lambda g, i: (g, i, 0)),
            pl.BlockSpec((NC, RB, 16), lambda g, i: (0, i, 0)),
        ],
        out_specs=[
            pl.BlockSpec((RB, D), lambda g, i: (i, g)),
            pl.BlockSpec((1, D), lambda g, i: (0, g)),
            pl.BlockSpec((1, D), lambda g, i: (0, g)),
        ],
        out_shape=[
            jax.ShapeDtypeStruct((N, HIN), F32),
            jax.ShapeDtypeStruct((1, HIN), F32),
            jax.ShapeDtypeStruct((1, HIN), F32),
        ],
    )(s, gacc, den)


def _d2_body(cs_ref, cq_ref, g_ref, b_ref, w_ref, sb_ref, wp_ref, bp_ref):
    mu = cs_ref[...] * (1.0 / N)
    var = cq_ref[...] * (1.0 / N) - mu * mu
    a = g_ref[...] * lax.rsqrt(var + 1e-5)
    c = b_ref[...] - mu * a
    w = w_ref[...]
    wp_ref[...] = w * jnp.transpose(a)
    bp_ref[...] = jnp.dot(c, w, preferred_element_type=F32, precision=lax.Precision.HIGHEST) + sb_ref[...]


def _d2_call(cs, cq, g, b, wcat1, sb1pad):
    return pl.pallas_call(
        _d2_body,
        out_shape=[
            jax.ShapeDtypeStruct((HIN, 516), F32),
            jax.ShapeDtypeStruct((1, 516), F32),
        ],
    )(cs, cq, g, b, wcat1, sb1pad)


def _d3_body(y_ref, wp_ref, bp_ref, hs_ref, k_ref, q_ref):
    y = jnp.dot(y_ref[...], wp_ref[...], preferred_element_type=F32, precision=lax.Precision.HIGHEST) + bp_ref[...]
    k = y[:, 0:KQ]
    q = y[:, KQ:2 * KQ]
    hs_ref[...] = jnp.concatenate(
        [y[:, 2 * KQ:2 * KQ + 4], jnp.zeros((y.shape[0], 12), F32)], axis=1)
    k_ref[...] = (k / (jnp.sqrt(jnp.sum(k * k, axis=1, keepdims=True)) + 1e-8)).astype(BF16)
    q_ref[...] = (q / (jnp.sqrt(jnp.sum(q * q, axis=1, keepdims=True)) + 1e-8)).astype(BF16)


def _d3_call(y0, wp1, bp1):
    nsteps = N // RB
    return pl.pallas_call(
        _d3_body,
        grid=(nsteps,),
        in_specs=[
            pl.BlockSpec((RB, HIN), lambda i: (i, 0)),
            pl.BlockSpec((HIN, 516), lambda i: (0, 0)),
            pl.BlockSpec((1, 516), lambda i: (0, 0)),
        ],
        out_specs=[
            pl.BlockSpec((RB, 16), lambda i: (i, 0)),
            pl.BlockSpec((RB, KQ), lambda i: (i, 0)),
            pl.BlockSpec((RB, KQ), lambda i: (i, 0)),
        ],
        out_shape=[
            jax.ShapeDtypeStruct((N, 16), F32),
            jax.ShapeDtypeStruct((N, KQ), BF16),
            jax.ShapeDtypeStruct((N, KQ), BF16),
        ],
    )(y0, wp1, bp1)


def _e_compute(kb, qb, hsb, mbuf, rows):
    sh16 = jnp.full((16,), 16, jnp.int32)
    mhi = jnp.full((16,), -65536, jnp.int32)

    def col_body(j, acc):
        colj = jnp.full((16,), 0, jnp.int32) + j
        ck = plsc.load_gather(kb, [rows, colj])
        cq = plsc.load_gather(qb, [rows, colj])
        ke, ko = _bf_pair(ck, sh16, mhi)
        qe, qo = _bf_pair(cq, sh16, mhi)
        return acc + ke * qe + ko * qo

    acc = lax.fori_loop(0, KQI, col_body, jnp.zeros((16,), F32), unroll=8)
    pv = jnp.exp(acc * (1.0 / 16.0))
    h0 = plsc.load_gather(hsb, [rows, jnp.full((16,), 0, jnp.int32)])
    h1 = plsc.load_gather(hsb, [rows, jnp.full((16,), 1, jnp.int32)])
    plsc.store_scatter(mbuf, [rows, jnp.full((16,), 0, jnp.int32)], pv * h0)
    plsc.store_scatter(mbuf, [rows, jnp.full((16,), 1, jnp.int32)], pv * h1)
    plsc.store_scatter(mbuf, [rows, jnp.full((16,), 2, jnp.int32)], pv)


def _e_body(src_ref, dst_ref, k_ref, q_ref, hs_ref, z_ref, ep_ref,
            ebs2, ebd2, kb0, qb0, hsb0, kb1, qb1, hsb1, mbuf0, mbuf1, macc,
            gs0, gs1, gs2, gs3, gs4, gs5, ss0, ss1):
    ci = lax.axis_index("c")
    si = lax.axis_index("s")
    wid = si * NC + ci

    @pl.when(si == 0)
    def _():
        pltpu.sync_copy(z_ref, macc)

    zero16 = jnp.zeros((16,), F32)
    for e in range(16):
        mbuf0[e, :] = zero16
        mbuf1[e, :] = zero16

    rowbase, nb = _worker_rows(wid)
    pltpu.sync_copy(src_ref.at[pl.ds(rowbase, EB_LO // 16), :],
                    ebs2.at[pl.ds(0, EB_LO // 16), :])
    pltpu.sync_copy(dst_ref.at[pl.ds(rowbase, EB_LO // 16), :],
                    ebd2.at[pl.ds(0, EB_LO // 16), :])

    @pl.when(wid < 16)
    def _():
        pltpu.sync_copy(src_ref.at[pl.ds(rowbase + EB_LO // 16, 1), :],
                        ebs2.at[pl.ds(EB_LO // 16, 1), :])
        pltpu.sync_copy(dst_ref.at[pl.ds(rowbase + EB_LO // 16, 1), :],
                        ebd2.at[pl.ds(EB_LO // 16, 1), :])

    plsc.subcore_barrier()
    rows = lax.iota(jnp.int32, 16)

    def pair_body(k, carry):
        b0 = 2 * k
        b1 = 2 * k + 1
        cka = pltpu.async_copy(k_ref.at[ebs2.at[b0]], kb0, gs0)
        cqa = pltpu.async_copy(q_ref.at[ebd2.at[b0]], qb0, gs1)
        cha = pltpu.async_copy(hs_ref.at[ebs2.at[b0]], hsb0, gs2)
        ckb = pltpu.async_copy(k_ref.at[ebs2.at[b1]], kb1, gs3)
        cqb = pltpu.async_copy(q_ref.at[ebd2.at[b1]], qb1, gs4)
        chb = pltpu.async_copy(hs_ref.at[ebs2.at[b1]], hsb1, gs5)
        cka.wait()
        cqa.wait()
        cha.wait()
        _e_compute(kb0, qb0, hsb0, mbuf0, rows)
        sca = pltpu.async_copy(mbuf0, macc.at[ebd2.at[b0]], ss0, add=True)
        ckb.wait()
        cqb.wait()
        chb.wait()
        _e_compute(kb1, qb1, hsb1, mbuf1, rows)
        scb = pltpu.async_copy(mbuf1, macc.at[ebd2.at[b1]], ss1, add=True)
        sca.wait()
        scb.wait()
        return carry

    lax.fori_loop(0, nb // 2, pair_body, 0)

    @pl.when(nb % 2 == 1)
    def _():
        bt = nb - 1
        pltpu.async_copy(k_ref.at[ebs2.at[bt]], kb0, gs0).wait()
        pltpu.async_copy(q_ref.at[ebd2.at[bt]], qb0, gs1).wait()
        pltpu.async_copy(hs_ref.at[ebs2.at[bt]], hsb0, gs2).wait()
        _e_compute(kb0, qb0, hsb0, mbuf0, rows)
        pltpu.sync_copy(mbuf0, macc.at[ebd2.at[bt]], add=True)

    plsc.subcore_barrier()

    @pl.when(si == 0)
    def _():
        pltpu.sync_copy(macc, ep_ref.at[ci])


def _e_call(src, dst, k1, q1, hs1, zn16):
    f = functools.partial(
        pl.kernel,
        out_type=jax.ShapeDtypeStruct((NC, N, 16), F32),
        mesh=_mesh,
        compiler_params=_SC_PARAMS,
        scratch_types=[
            pltpu.VMEM((EB_HI // 16, 16), jnp.int32),
            pltpu.VMEM((EB_HI // 16, 16), jnp.int32),
            pltpu.VMEM((16, KQI), jnp.int32),
            pltpu.VMEM((16, KQI), jnp.int32),
            pltpu.VMEM((16, 16), F32),
            pltpu.VMEM((16, KQI), jnp.int32),
            pltpu.VMEM((16, KQI), jnp.int32),
            pltpu.VMEM((16, 16), F32),
            pltpu.VMEM((16, 16), F32),
            pltpu.VMEM((16, 16), F32),
            pltpu.VMEM_SHARED((N, 16), F32),
            pltpu.SemaphoreType.DMA,
            pltpu.SemaphoreType.DMA,
            pltpu.SemaphoreType.DMA,
            pltpu.SemaphoreType.DMA,
            pltpu.SemaphoreType.DMA,
            pltpu.SemaphoreType.DMA,
            pltpu.SemaphoreType.DMA,
            pltpu.SemaphoreType.DMA,
        ],
    )(_e_body)
    return f(src.reshape(E // 16, 16), dst.reshape(E // 16, 16), k1, q1, hs1,
             zn16)


def _f_body(ep_ref, hs_ref, o_ref):
    m = ep_ref[0] + ep_ref[1]
    rec = 1.0 / (m[:, 2:3] + 1e-16)
    o_ref[...] = jnp.maximum(hs_ref[:, 2:4] + m[:, 0:2] * rec, 0.0)


def _f_call(ep, hs1):
    return pl.pallas_call(
        _f_body,
        out_shape=jax.ShapeDtypeStruct((N, 2), F32),
    )(ep, hs1)


def kernel(x, edge_index, W0, Wk0, Wq0, SW0, Sb0, g0, b0,
           W1, Wk1, Wq1, SW1, Sb1, g1, b1):
    src = edge_index[0]
    dst = edge_index[1]
    zn16 = jnp.zeros((N, 16), F32)
    zn128 = jnp.zeros((N, D), F32)

    wcat = jnp.concatenate([W0, Wk0, Wq0, SW0], axis=1)
    sbpad = jnp.concatenate([jnp.zeros((3 * HIN,), F32), Sb0]).reshape(1, 4 * HIN)
    wp, bp = _a1_call(x, wcat, g0.reshape(1, D), b0.reshape(1, D), sbpad)
    h, kn, qn, s = _a2_call(x, wp, bp)

    kn_i = lax.bitcast_convert_type(kn.reshape(N, HEADS * KQI, 2), jnp.int32)
    qn_i = lax.bitcast_convert_type(qn.reshape(N, HEADS * KQI, 2), jnp.int32)
    h8_i = lax.bitcast_convert_type(h.reshape(N * NGRP, DI, 2), jnp.int32)
    p, den = _b_call(src, dst, kn_i, qn_i, zn16)
    gacc = _c_call(src, dst, p, h8_i, zn128)

    y0, cs, cq = _d_call(s, gacc, den)
    wcat1 = jnp.concatenate([Wk1, Wq1, W1, SW1], axis=1)
    sb1pad = jnp.concatenate([jnp.zeros((2 * KQ + 2,), F32), Sb1]).reshape(1, 516)
    wp1, bp1 = _d2_call(cs, cq, g1.reshape(1, HIN), b1.reshape(1, HIN),
                        wcat1, sb1pad)
    hs1, k1, q1 = _d3_call(y0, wp1, bp1)

    k1_i = lax.bitcast_convert_type(k1.reshape(N, KQI, 2), jnp.int32)
    q1_i = lax.bitcast_convert_type(q1.reshape(N, KQI, 2), jnp.int32)
    ep = _e_call(src, dst, k1_i, q1_i, hs1, zn16)
    return _f_call(ep, hs1)


# speedup vs baseline: 2.5546x; 1.0687x over previous
import functools
import jax
import jax.numpy as jnp
from jax import lax
from jax.experimental import pallas as pl
from jax.experimental.pallas import tpu as pltpu
from jax.experimental.pallas import tpu_sc as plsc

N = 10000
E = 160000
D = 128
HID = 256
HEADS = 4
KQ = 256
HIN = HEADS * HID
F32 = jnp.float32
BF16 = jnp.bfloat16
KQI = KQ // 2
KQW = KQ // 4
DI = D // 2

NC = 2
NS = 16
NW = NC * NS

RB = 1000
EPW = E // NW
BB = 40
EPC = E // NS
BC = 16
NGRP = 8
GPC = NGRP // NC

_mesh = plsc.VectorSubcoreMesh(core_axis_name="c", subcore_axis_name="s")
_SC_PARAMS = pltpu.CompilerParams(use_tc_tiling_on_sc=False,
                                  needs_layout_passes=False)


def _a1_body(x_ref, w_ref, g_ref, b_ref, sb_ref, wp_ref, bp_ref):
    x = x_ref[...]
    mu = jnp.mean(x, axis=0, keepdims=True)
    var = jnp.mean(x * x, axis=0, keepdims=True) - mu * mu
    a = g_ref[...] * lax.rsqrt(var + 1e-5)
    c = b_ref[...] - mu * a
    w = w_ref[...]
    wp_ref[...] = w * jnp.transpose(a)
    bp_ref[...] = jnp.dot(c, w, preferred_element_type=F32, precision=lax.Precision.HIGHEST) + sb_ref[...]


def _a1_call(x, wcat, g, b, sbpad):
    return pl.pallas_call(
        _a1_body,
        out_shape=[
            jax.ShapeDtypeStruct((D, 4 * HIN), F32),
            jax.ShapeDtypeStruct((1, 4 * HIN), F32),
        ],
    )(x, wcat, g, b, sbpad)


def _a2_body(x_ref, wp_ref, bp_ref, h_ref, kn_ref, qn_ref, s_ref):
    y = jnp.dot(x_ref[...], wp_ref[...], preferred_element_type=F32, precision=lax.Precision.HIGHEST) + bp_ref[...]
    h_ref[...] = y[:, 0:HIN].astype(BF16)
    s_ref[...] = y[:, 3 * HIN:4 * HIN]
    for lo, ref in ((HIN, kn_ref), (2 * HIN, qn_ref)):
        for hh in range(HEADS):
            ch = y[:, lo + hh * KQ: lo + (hh + 1) * KQ]
            nrm = jnp.sqrt(jnp.sum(ch * ch, axis=1, keepdims=True)) + 1e-8
            ref[:, hh * KQ:(hh + 1) * KQ] = jnp.round(
                ch / nrm * 127.0).astype(jnp.int8)


def _a2_call(x, wp, bp):
    nsteps = N // RB
    return pl.pallas_call(
        _a2_body,
        grid=(nsteps,),
        in_specs=[
            pl.BlockSpec((RB, D), lambda i: (i, 0)),
            pl.BlockSpec((D, 4 * HIN), lambda i: (0, 0)),
            pl.BlockSpec((1, 4 * HIN), lambda i: (0, 0)),
        ],
        out_specs=[
            pl.BlockSpec((RB, HIN), lambda i: (i, 0)),
            pl.BlockSpec((RB, HIN), lambda i: (i, 0)),
            pl.BlockSpec((RB, HIN), lambda i: (i, 0)),
            pl.BlockSpec((RB, HIN), lambda i: (i, 0)),
        ],
        out_shape=[
            jax.ShapeDtypeStruct((N, HIN), BF16),
            jax.ShapeDtypeStruct((N, HIN), jnp.int8),
            jax.ShapeDtypeStruct((N, HIN), jnp.int8),
            jax.ShapeDtypeStruct((N, HIN), F32),
        ],
    )(x, wp, bp)


EB_LO = 4992
EB_HI = 5008


def _worker_rows(wid):
    rowbase = wid * (EB_LO // 16) + jnp.minimum(wid, 16)
    nb = jnp.where(wid < 16, EB_HI // 16, EB_LO // 16)
    return rowbase, nb


def _bf_pair(v, sh16, mhi):
    lo = lax.bitcast_convert_type(lax.shift_left(v, sh16), F32)
    hi = lax.bitcast_convert_type(v & mhi, F32)
    return lo, hi


def _i8_bytes(v, s8, s16, s24):
    b0 = lax.shift_right_arithmetic(lax.shift_left(v, s24), s24)
    b1 = lax.shift_right_arithmetic(lax.shift_left(v, s16), s24)
    b2 = lax.shift_right_arithmetic(lax.shift_left(v, s8), s24)
    b3 = lax.shift_right_arithmetic(v, s24)
    return b0, b1, b2, b3


_I8SC = 1.0 / (127.0 * 127.0 * 16.0)


def _b_compute(kb, qb, pbuf, pall, b, rows):
    s8 = jnp.full((16,), 8, jnp.int32)
    s16 = jnp.full((16,), 16, jnp.int32)
    s24 = jnp.full((16,), 24, jnp.int32)

    def col_body(j, accs):
        colj = jnp.full((16,), 0, jnp.int32) + j
        new = []
        for hh in range(HEADS):
            cols = colj + (hh * KQW)
            ck = plsc.load_gather(kb, [rows, cols])
            cq = plsc.load_gather(qb, [rows, cols])
            k0, k1, k2, k3 = _i8_bytes(ck, s8, s16, s24)
            q0, q1, q2, q3 = _i8_bytes(cq, s8, s16, s24)
            new.append(accs[hh] + (k0 * q0 + k1 * q1) + (k2 * q2 + k3 * q3))
        return tuple(new)

    accs = lax.fori_loop(0, KQW, col_body,
                         tuple(jnp.zeros((16,), jnp.int32) for _ in range(HEADS)),
                         unroll=4)
    for hh in range(HEADS):
        pv = jnp.exp(accs[hh].astype(F32) * _I8SC)
        pall[hh, pl.ds(b * 16, 16)] = pv
        plsc.store_scatter(pbuf, [rows, jnp.full((16,), hh, jnp.int32)], pv)


def _b_body(src_ref, dst_ref, kn_ref, qn_ref, z_ref, p_ref, den_ref,
            ebs2, ebd2, kb0, qb0, kb1, qb1, pbuf0, pbuf1, pall, dacc,
            gs0, gs1, gs2, gs3, ss0, ss1):
    ci = lax.axis_index("c")
    si = lax.axis_index("s")
    wid = si * NC + ci

    @pl.when(si == 0)
    def _():
        pltpu.sync_copy(z_ref, dacc)

    zero16 = jnp.zeros((16,), F32)
    for e in range(16):
        pbuf0[e, :] = zero16
        pbuf1[e, :] = zero16

    rowbase, nb = _worker_rows(wid)
    pltpu.sync_copy(src_ref.at[pl.ds(rowbase, EB_LO // 16), :],
                    ebs2.at[pl.ds(0, EB_LO // 16), :])
    pltpu.sync_copy(dst_ref.at[pl.ds(rowbase, EB_LO // 16), :],
                    ebd2.at[pl.ds(0, EB_LO // 16), :])

    @pl.when(wid < 16)
    def _():
        pltpu.sync_copy(src_ref.at[pl.ds(rowbase + EB_LO // 16, 1), :],
                        ebs2.at[pl.ds(EB_LO // 16, 1), :])
        pltpu.sync_copy(dst_ref.at[pl.ds(rowbase + EB_LO // 16, 1), :],
                        ebd2.at[pl.ds(EB_LO // 16, 1), :])

    plsc.subcore_barrier()
    rows = lax.iota(jnp.int32, 16)

    def pair_body(k, carry):
        b0 = 2 * k
        b1 = 2 * k + 1
        cka = pltpu.async_copy(kn_ref.at[ebs2.at[b0]], kb0, gs0)
        cqa = pltpu.async_copy(qn_ref.at[ebd2.at[b0]], qb0, gs1)
        ckb = pltpu.async_copy(kn_ref.at[ebs2.at[b1]], kb1, gs2)
        cqb = pltpu.async_copy(qn_ref.at[ebd2.at[b1]], qb1, gs3)
        cka.wait()
        cqa.wait()
        _b_compute(kb0, qb0, pbuf0, pall, b0, rows)
        sca = pltpu.async_copy(pbuf0, dacc.at[ebd2.at[b0]], ss0, add=True)
        ckb.wait()
        cqb.wait()
        _b_compute(kb1, qb1, pbuf1, pall, b1, rows)
        scb = pltpu.async_copy(pbuf1, dacc.at[ebd2.at[b1]], ss1, add=True)
        sca.wait()
        scb.wait()
        return carry

    lax.fori_loop(0, nb // 2, pair_body, 0)

    @pl.when(nb % 2 == 1)
    def _():
        bt = nb - 1
        pltpu.async_copy(kn_ref.at[ebs2.at[bt]], kb0, gs0).wait()
        pltpu.async_copy(qn_ref.at[ebd2.at[bt]], qb0, gs1).wait()
        _b_compute(kb0, qb0, pbuf0, pall, bt, rows)
        pltpu.sync_copy(pbuf0, dacc.at[ebd2.at[bt]], add=True)

    base_w = rowbase * 16

    @pl.when(wid < 16)
    def _():
        for hh in range(HEADS):
            pltpu.sync_copy(pall.at[hh, pl.ds(0, EB_HI)],
                            p_ref.at[pl.ds(hh * E + base_w, EB_HI)])

    @pl.when(wid >= 16)
    def _():
        for hh in range(HEADS):
            pltpu.sync_copy(pall.at[hh, pl.ds(0, EB_LO)],
                            p_ref.at[pl.ds(hh * E + base_w, EB_LO)])

    plsc.subcore_barrier()

    @pl.when(si == 0)
    def _():
        pltpu.sync_copy(dacc, den_ref.at[ci])


def _b_call(src, dst, kn, qn, zn16):
    f = functools.partial(
        pl.kernel,
        out_type=[
            jax.ShapeDtypeStruct((HEADS * E,), F32),
            jax.ShapeDtypeStruct((NC, N, 16), F32),
        ],
        mesh=_mesh,
        compiler_params=_SC_PARAMS,
        scratch_types=[
            pltpu.VMEM((EB_HI // 16, 16), jnp.int32),
            pltpu.VMEM((EB_HI // 16, 16), jnp.int32),
            pltpu.VMEM((16, HEADS * KQW), jnp.int32),
            pltpu.VMEM((16, HEADS * KQW), jnp.int32),
            pltpu.VMEM((16, HEADS * KQW), jnp.int32),
            pltpu.VMEM((16, HEADS * KQW), jnp.int32),
            pltpu.VMEM((16, 16), F32),
            pltpu.VMEM((16, 16), F32),
            pltpu.VMEM((HEADS, EB_HI), F32),
            pltpu.VMEM_SHARED((N, 16), F32),
            pltpu.SemaphoreType.DMA,
            pltpu.SemaphoreType.DMA,
            pltpu.SemaphoreType.DMA,
            pltpu.SemaphoreType.DMA,
            pltpu.SemaphoreType.DMA,
            pltpu.SemaphoreType.DMA,
        ],
    )(_b_body)
    return f(src.reshape(E // 16, 16), dst.reshape(E // 16, 16), kn, qn, zn16)


CB = 80
CROWS = E // CB // NS
CCH = 25


def _c_scale(hb, ob, pb2, bi, rows_list):
    pvs = [pb2[bi, pl.ds(r * 16, 16)] for r in range(5)]
    sh16 = jnp.full((16,), 16, jnp.int32)
    mhi = jnp.full((16,), -65536, jnp.int32)
    one = jnp.full((16,), 1, jnp.int32)

    def col_body(c, c2):
        cc = jnp.full((16,), 0, jnp.int32) + c
        ce = cc + cc
        for r in range(5):
            v = plsc.load_gather(hb, [rows_list[r], cc])
            lo, hi = _bf_pair(v, sh16, mhi)
            plsc.store_scatter(ob, [rows_list[r], ce], lo * pvs[r])
            plsc.store_scatter(ob, [rows_list[r], ce + one], hi * pvs[r])
        return c2

    lax.fori_loop(0, DI, col_body, 0, unroll=4)


def _c_body(src_ref, dst_ref, p_ref, h8_ref, z_ref, g_out,
            ebd2, pb2, idx82, hb0, hb1, ob0, ob1, gacc,
            sem0, sem1, ssem0, ssem1):
    ci = lax.axis_index("c")
    si = lax.axis_index("s")
    rb = si * CROWS
    rows_list = [lax.iota(jnp.int32, 16) + r * 16 for r in range(5)]
    for gi in range(GPC):
        g = ci * GPC + gi
        head = g // 2

        @pl.when(si == 0)
        def _():
            pltpu.sync_copy(z_ref, gacc)

        plsc.subcore_barrier()

        def chunk_body(ch, c0):
            crb = rb + ch * CCH
            pltpu.sync_copy(src_ref.at[pl.ds(crb, CCH), :], idx82)
            pltpu.sync_copy(dst_ref.at[pl.ds(crb, CCH), :], ebd2)
            pltpu.sync_copy(p_ref.at[pl.ds(head * (NS * CROWS) + crb, CCH), :],
                            pb2)

            def idx_body(r, c2):
                for cc in range(5):
                    sl = pl.ds(cc * 16, 16)
                    idx82[r, sl] = idx82[r, sl] * 8 + g
                return c2

            lax.fori_loop(0, CCH, idx_body, 0)

            def pair_body(k, c2):
                b0 = 2 * k
                b1 = 2 * k + 1
                cpa = pltpu.async_copy(h8_ref.at[idx82.at[b0]], hb0, sem0)
                cpb = pltpu.async_copy(h8_ref.at[idx82.at[b1]], hb1, sem1)
                cpa.wait()
                _c_scale(hb0, ob0, pb2, b0, rows_list)
                sca = pltpu.async_copy(ob0, gacc.at[ebd2.at[b0]], ssem0,
                                       add=True)
                cpb.wait()
                _c_scale(hb1, ob1, pb2, b1, rows_list)
                scb = pltpu.async_copy(ob1, gacc.at[ebd2.at[b1]], ssem1,
                                       add=True)
                sca.wait()
                scb.wait()
                return c2

            lax.fori_loop(0, CCH // 2, pair_body, 0)
            bt = CCH - 1
            pltpu.async_copy(h8_ref.at[idx82.at[bt]], hb0, sem0).wait()
            _c_scale(hb0, ob0, pb2, bt, rows_list)
            pltpu.sync_copy(ob0, gacc.at[ebd2.at[bt]], add=True)
            return c0

        lax.fori_loop(0, CROWS // CCH, chunk_body, 0)
        plsc.subcore_barrier()

        @pl.when(si == 0)
        def _():
            pltpu.sync_copy(gacc, g_out.at[g])

        plsc.subcore_barrier()


def _c_call(src, dst, p, h8, zn128):
    f = functools.partial(
        pl.kernel,
        out_type=jax.ShapeDtypeStruct((NGRP, N, D), F32),
        mesh=_mesh,
        compiler_params=_SC_PARAMS,
        scratch_types=[
            pltpu.VMEM((CCH, CB), jnp.int32),
            pltpu.VMEM((CCH, CB), F32),
            pltpu.VMEM((CCH, CB), jnp.int32),
            pltpu.VMEM((CB, DI), jnp.int32),
            pltpu.VMEM((CB, DI), jnp.int32),
            pltpu.VMEM((CB, D), F32),
            pltpu.VMEM((CB, D), F32),
            pltpu.VMEM_SHARED((N, D), F32),
            pltpu.SemaphoreType.DMA,
            pltpu.SemaphoreType.DMA,
            pltpu.SemaphoreType.DMA,
            pltpu.SemaphoreType.DMA,
        ],
    )(_c_body)
    return f(src.reshape(E // CB, CB), dst.reshape(E // CB, CB),
             p.reshape(HEADS * (E // CB), CB), h8, zn128)


def _d_body(s_ref, g_ref, den_ref, y_ref, cs_ref, cq_ref):
    g = pl.program_id(0)
    i = pl.program_id(1)
    head = g // 2
    den = den_ref[0] + den_ref[1]
    onehot = (lax.broadcasted_iota(jnp.int32, (1, 16), 1) == head).astype(F32)
    dh = jnp.sum(den * onehot, axis=1, keepdims=True)
    rec = 1.0 / (dh + 1e-16)
    y = jnp.maximum(s_ref[...] + g_ref[0] * rec, 0.0)
    y_ref[...] = y

    @pl.when(i == 0)
    def _():
        cs_ref[...] = jnp.zeros_like(cs_ref)
        cq_ref[...] = jnp.zeros_like(cq_ref)

    cs_ref[...] += jnp.sum(y, axis=0, keepdims=True)
    cq_ref[...] += jnp.sum(y * y, axis=0, keepdims=True)


def _d_call(s, gacc, den):
    nsteps = N // RB
    return pl.pallas_call(
        _d_body,
        grid=(NGRP, nsteps),
        in_specs=[
            pl.BlockSpec((RB, D), lambda g, i: (i, g)),
            pl.BlockSpec((1, RB, D), lambda g, i: (g, i, 0)),
            pl.BlockSpec((NC, RB, 16), lambda g, i: (0, i, 0)),
        ],
        out_specs=[
            pl.BlockSpec((RB, D), lambda g, i: (i, g)),
            pl.BlockSpec((1, D), lambda g, i: (0, g)),
            pl.BlockSpec((1, D), lambda g, i: (0, g)),
        ],
        out_shape=[
            jax.ShapeDtypeStruct((N, HIN), F32),
            jax.ShapeDtypeStruct((1, HIN), F32),
            jax.ShapeDtypeStruct((1, HIN), F32),
        ],
    )(s, gacc, den)


def _d2_body(cs_ref, cq_ref, g_ref, b_ref, w_ref, sb_ref, wp_ref, bp_ref):
    mu = cs_ref[...] * (1.0 / N)
    var = cq_ref[...] * (1.0 / N) - mu * mu
    a = g_ref[...] * lax.rsqrt(var + 1e-5)
    c = b_ref[...] - mu * a
    w = w_ref[...]
    wp_ref[...] = w * jnp.transpose(a)
    bp_ref[...] = jnp.dot(c, w, preferred_element_type=F32, precision=lax.Precision.HIGHEST) + sb_ref[...]


def _d2_call(cs, cq, g, b, wcat1, sb1pad):
    return pl.pallas_call(
        _d2_body,
        out_shape=[
            jax.ShapeDtypeStruct((HIN, 516), F32),
            jax.ShapeDtypeStruct((1, 516), F32),
        ],
    )(cs, cq, g, b, wcat1, sb1pad)


def _d3_body(y_ref, wp_ref, bp_ref, hs_ref, k_ref, q_ref):
    y = jnp.dot(y_ref[...], wp_ref[...], preferred_element_type=F32, precision=lax.Precision.HIGHEST) + bp_ref[...]
    k = y[:, 0:KQ]
    q = y[:, KQ:2 * KQ]
    hs_ref[...] = jnp.concatenate(
        [y[:, 2 * KQ:2 * KQ + 4], jnp.zeros((y.shape[0], 12), F32)], axis=1)
    k_ref[...] = jnp.round(k / (jnp.sqrt(jnp.sum(k * k, axis=1, keepdims=True)) + 1e-8) * 127.0).astype(jnp.int8)
    q_ref[...] = jnp.round(q / (jnp.sqrt(jnp.sum(q * q, axis=1, keepdims=True)) + 1e-8) * 127.0).astype(jnp.int8)


def _d3_call(y0, wp1, bp1):
    nsteps = N // RB
    return pl.pallas_call(
        _d3_body,
        grid=(nsteps,),
        in_specs=[
            pl.BlockSpec((RB, HIN), lambda i: (i, 0)),
            pl.BlockSpec((HIN, 516), lambda i: (0, 0)),
            pl.BlockSpec((1, 516), lambda i: (0, 0)),
        ],
        out_specs=[
            pl.BlockSpec((RB, 16), lambda i: (i, 0)),
            pl.BlockSpec((RB, KQ), lambda i: (i, 0)),
            pl.BlockSpec((RB, KQ), lambda i: (i, 0)),
        ],
        out_shape=[
            jax.ShapeDtypeStruct((N, 16), F32),
            jax.ShapeDtypeStruct((N, KQ), jnp.int8),
            jax.ShapeDtypeStruct((N, KQ), jnp.int8),
        ],
    )(y0, wp1, bp1)


def _e_compute(kb, qb, hsb, mbuf, rows):
    s8 = jnp.full((16,), 8, jnp.int32)
    s16 = jnp.full((16,), 16, jnp.int32)
    s24 = jnp.full((16,), 24, jnp.int32)

    def col_body(j, acc):
        colj = jnp.full((16,), 0, jnp.int32) + j
        ck = plsc.load_gather(kb, [rows, colj])
        cq = plsc.load_gather(qb, [rows, colj])
        k0, k1, k2, k3 = _i8_bytes(ck, s8, s16, s24)
        q0, q1, q2, q3 = _i8_bytes(cq, s8, s16, s24)
        return acc + (k0 * q0 + k1 * q1) + (k2 * q2 + k3 * q3)

    acc = lax.fori_loop(0, KQW, col_body, jnp.zeros((16,), jnp.int32),
                        unroll=8)
    pv = jnp.exp(acc.astype(F32) * _I8SC)
    h0 = plsc.load_gather(hsb, [rows, jnp.full((16,), 0, jnp.int32)])
    h1 = plsc.load_gather(hsb, [rows, jnp.full((16,), 1, jnp.int32)])
    plsc.store_scatter(mbuf, [rows, jnp.full((16,), 0, jnp.int32)], pv * h0)
    plsc.store_scatter(mbuf, [rows, jnp.full((16,), 1, jnp.int32)], pv * h1)
    plsc.store_scatter(mbuf, [rows, jnp.full((16,), 2, jnp.int32)], pv)


def _e_body(src_ref, dst_ref, k_ref, q_ref, hs_ref, z_ref, ep_ref,
            ebs2, ebd2, kb0, qb0, hsb0, kb1, qb1, hsb1, mbuf0, mbuf1, macc,
            gs0, gs1, gs2, gs3, gs4, gs5, ss0, ss1):
    ci = lax.axis_index("c")
    si = lax.axis_index("s")
    wid = si * NC + ci

    @pl.when(si == 0)
    def _():
        pltpu.sync_copy(z_ref, macc)

    zero16 = jnp.zeros((16,), F32)
    for e in range(16):
        mbuf0[e, :] = zero16
        mbuf1[e, :] = zero16

    rowbase, nb = _worker_rows(wid)
    pltpu.sync_copy(src_ref.at[pl.ds(rowbase, EB_LO // 16), :],
                    ebs2.at[pl.ds(0, EB_LO // 16), :])
    pltpu.sync_copy(dst_ref.at[pl.ds(rowbase, EB_LO // 16), :],
                    ebd2.at[pl.ds(0, EB_LO // 16), :])

    @pl.when(wid < 16)
    def _():
        pltpu.sync_copy(src_ref.at[pl.ds(rowbase + EB_LO // 16, 1), :],
                        ebs2.at[pl.ds(EB_LO // 16, 1), :])
        pltpu.sync_copy(dst_ref.at[pl.ds(rowbase + EB_LO // 16, 1), :],
                        ebd2.at[pl.ds(EB_LO // 16, 1), :])

    plsc.subcore_barrier()
    rows = lax.iota(jnp.int32, 16)

    def pair_body(k, carry):
        b0 = 2 * k
        b1 = 2 * k + 1
        cka = pltpu.async_copy(k_ref.at[ebs2.at[b0]], kb0, gs0)
        cqa = pltpu.async_copy(q_ref.at[ebd2.at[b0]], qb0, gs1)
        cha = pltpu.async_copy(hs_ref.at[ebs2.at[b0]], hsb0, gs2)
        ckb = pltpu.async_copy(k_ref.at[ebs2.at[b1]], kb1, gs3)
        cqb = pltpu.async_copy(q_ref.at[ebd2.at[b1]], qb1, gs4)
        chb = pltpu.async_copy(hs_ref.at[ebs2.at[b1]], hsb1, gs5)
        cka.wait()
        cqa.wait()
        cha.wait()
        _e_compute(kb0, qb0, hsb0, mbuf0, rows)
        sca = pltpu.async_copy(mbuf0, macc.at[ebd2.at[b0]], ss0, add=True)
        ckb.wait()
        cqb.wait()
        chb.wait()
        _e_compute(kb1, qb1, hsb1, mbuf1, rows)
        scb = pltpu.async_copy(mbuf1, macc.at[ebd2.at[b1]], ss1, add=True)
        sca.wait()
        scb.wait()
        return carry

    lax.fori_loop(0, nb // 2, pair_body, 0)

    @pl.when(nb % 2 == 1)
    def _():
        bt = nb - 1
        pltpu.async_copy(k_ref.at[ebs2.at[bt]], kb0, gs0).wait()
        pltpu.async_copy(q_ref.at[ebd2.at[bt]], qb0, gs1).wait()
        pltpu.async_copy(hs_ref.at[ebs2.at[bt]], hsb0, gs2).wait()
        _e_compute(kb0, qb0, hsb0, mbuf0, rows)
        pltpu.sync_copy(mbuf0, macc.at[ebd2.at[bt]], add=True)

    plsc.subcore_barrier()

    @pl.when(si == 0)
    def _():
        pltpu.sync_copy(macc, ep_ref.at[ci])


def _e_call(src, dst, k1, q1, hs1, zn16):
    f = functools.partial(
        pl.kernel,
        out_type=jax.ShapeDtypeStruct((NC, N, 16), F32),
        mesh=_mesh,
        compiler_params=_SC_PARAMS,
        scratch_types=[
            pltpu.VMEM((EB_HI // 16, 16), jnp.int32),
            pltpu.VMEM((EB_HI // 16, 16), jnp.int32),
            pltpu.VMEM((16, KQW), jnp.int32),
            pltpu.VMEM((16, KQW), jnp.int32),
            pltpu.VMEM((16, 16), F32),
            pltpu.VMEM((16, KQW), jnp.int32),
            pltpu.VMEM((16, KQW), jnp.int32),
            pltpu.VMEM((16, 16), F32),
            pltpu.VMEM((16, 16), F32),
            pltpu.VMEM((16, 16), F32),
            pltpu.VMEM_SHARED((N, 16), F32),
            pltpu.SemaphoreType.DMA,
            pltpu.SemaphoreType.DMA,
            pltpu.SemaphoreType.DMA,
            pltpu.SemaphoreType.DMA,
            pltpu.SemaphoreType.DMA,
            pltpu.SemaphoreType.DMA,
            pltpu.SemaphoreType.DMA,
            pltpu.SemaphoreType.DMA,
        ],
    )(_e_body)
    return f(src.reshape(E // 16, 16), dst.reshape(E // 16, 16), k1, q1, hs1,
             zn16)


def _f_body(ep_ref, hs_ref, o_ref):
    m = ep_ref[0] + ep_ref[1]
    rec = 1.0 / (m[:, 2:3] + 1e-16)
    o_ref[...] = jnp.maximum(hs_ref[:, 2:4] + m[:, 0:2] * rec, 0.0)


def _f_call(ep, hs1):
    return pl.pallas_call(
        _f_body,
        out_shape=jax.ShapeDtypeStruct((N, 2), F32),
    )(ep, hs1)


def kernel(x, edge_index, W0, Wk0, Wq0, SW0, Sb0, g0, b0,
           W1, Wk1, Wq1, SW1, Sb1, g1, b1):
    src = edge_index[0]
    dst = edge_index[1]
    zn16 = jnp.zeros((N, 16), F32)
    zn128 = jnp.zeros((N, D), F32)

    wcat = jnp.concatenate([W0, Wk0, Wq0, SW0], axis=1)
    sbpad = jnp.concatenate([jnp.zeros((3 * HIN,), F32), Sb0]).reshape(1, 4 * HIN)
    wp, bp = _a1_call(x, wcat, g0.reshape(1, D), b0.reshape(1, D), sbpad)
    h, kn, qn, s = _a2_call(x, wp, bp)

    kn_i = lax.bitcast_convert_type(kn.reshape(N, HEADS * KQW, 4), jnp.int32)
    qn_i = lax.bitcast_convert_type(qn.reshape(N, HEADS * KQW, 4), jnp.int32)
    h8_i = lax.bitcast_convert_type(h.reshape(N * NGRP, DI, 2), jnp.int32)
    p, den = _b_call(src, dst, kn_i, qn_i, zn16)
    gacc = _c_call(src, dst, p, h8_i, zn128)

    y0, cs, cq = _d_call(s, gacc, den)
    wcat1 = jnp.concatenate([Wk1, Wq1, W1, SW1], axis=1)
    sb1pad = jnp.concatenate([jnp.zeros((2 * KQ + 2,), F32), Sb1]).reshape(1, 516)
    wp1, bp1 = _d2_call(cs, cq, g1.reshape(1, HIN), b1.reshape(1, HIN),
                        wcat1, sb1pad)
    hs1, k1, q1 = _d3_call(y0, wp1, bp1)

    k1_i = lax.bitcast_convert_type(k1.reshape(N, KQW, 4), jnp.int32)
    q1_i = lax.bitcast_convert_type(q1.reshape(N, KQW, 4), jnp.int32)
    ep = _e_call(src, dst, k1_i, q1_i, hs1, zn16)
    return _f_call(ep, hs1)
